# baseline (speedup 1.0000x reference)
import functools

import numpy as np
import jax
import jax.numpy as jnp
from jax import lax
from jax.experimental import pallas as pl
from jax.experimental.pallas import tpu as pltpu
from jax.experimental.pallas import tpu_sc as plsc

_N = 10000
_NPAD = 10240
_E = 320000
_HID = 128
_EDIM = 16
_TDIM = 64
_L = 4
_XW = 16

_NC = 2
_NS = 16
_NW = _NC * _NS
_EW = _E // _NW
_C = 80
_NCHUNK = _EW // _C
_RPT = _NPAD // _NS

_BE = 2000
_BN = 512
_NBN = _NPAD // _BN


def _silu(v):
    return v * (1.0 / (1.0 + jnp.exp(-v)))


def _sigmoid(v):
    return 1.0 / (1.0 + jnp.exp(-v))


@functools.partial(
    pl.kernel,
    out_type=(
        jax.ShapeDtypeStruct((_E, _HID), jnp.bfloat16),
        jax.ShapeDtypeStruct((_E, _HID), jnp.bfloat16),
        jax.ShapeDtypeStruct((_E, _XW), jnp.float32),
        jax.ShapeDtypeStruct((_E, _XW), jnp.float32),
    ),
    mesh=plsc.VectorSubcoreMesh(core_axis_name="c", subcore_axis_name="s"),
    scratch_types=[
        pltpu.VMEM((_C,), jnp.int32),
        pltpu.VMEM((_C,), jnp.int32),
        pltpu.VMEM((_C, _HID), jnp.bfloat16),
        pltpu.VMEM((_C, _HID), jnp.bfloat16),
        pltpu.VMEM((_C, _XW), jnp.float32),
        pltpu.VMEM((_C, _XW), jnp.float32),
        pltpu.SemaphoreType.DMA,
        pltpu.SemaphoreType.DMA,
        pltpu.SemaphoreType.DMA,
        pltpu.SemaphoreType.DMA,
    ],
    compiler_params=pltpu.CompilerParams(use_tc_tiling_on_sc=False),
    name="egnn_sc_gather",
)
def _sc_gather(hrp_h, hcp_h, xp_h, row_h, col_h, gr_h, gc_h, xr_h, xc_h,
               idxr, idxc, bufr, bufc, bufxr, bufxc, s1, s2, s3, s4):
    wid = lax.axis_index("s") * _NC + lax.axis_index("c")
    base0 = wid * _EW

    def body(i, carry):
        base = base0 + i * _C
        pltpu.sync_copy(row_h.at[pl.ds(base, _C)], idxr)
        pltpu.sync_copy(col_h.at[pl.ds(base, _C)], idxc)
        c1 = pltpu.async_copy(hrp_h.at[idxr], bufr, s1)
        c2 = pltpu.async_copy(hcp_h.at[idxc], bufc, s2)
        c3 = pltpu.async_copy(xp_h.at[idxr], bufxr, s3)
        c4 = pltpu.async_copy(xp_h.at[idxc], bufxc, s4)
        c1.wait()
        c2.wait()
        c3.wait()
        c4.wait()
        pltpu.sync_copy(bufr, gr_h.at[pl.ds(base, _C)])
        pltpu.sync_copy(bufc, gc_h.at[pl.ds(base, _C)])
        pltpu.sync_copy(bufxr, xr_h.at[pl.ds(base, _C)])
        pltpu.sync_copy(bufxc, xc_h.at[pl.ds(base, _C)])
        return carry

    lax.fori_loop(0, _NCHUNK, body, 0)


@functools.partial(
    pl.kernel,
    out_type=(
        jax.ShapeDtypeStruct((_NC, _NPAD, _HID), jnp.float32),
        jax.ShapeDtypeStruct((_NC, _NPAD, _XW), jnp.float32),
    ),
    mesh=plsc.VectorSubcoreMesh(core_axis_name="c", subcore_axis_name="s"),
    scratch_types=[
        pltpu.VMEM((_C,), jnp.int32),
        pltpu.VMEM((_C, _HID), jnp.float32),
        pltpu.VMEM((_C, _XW), jnp.float32),
        pltpu.VMEM_SHARED((_NPAD, _HID), jnp.float32),
        pltpu.VMEM_SHARED((_NPAD, _XW), jnp.float32),
    ],
    compiler_params=pltpu.CompilerParams(use_tc_tiling_on_sc=False),
    name="egnn_sc_scatter",
)
def _sc_scatter(ma_h, cu_h, row_h, z128_h, z16_h, hp_h, xp_h,
                idx, bufm, bufc, hacc, xacc):
    cid = lax.axis_index("c")
    sid = lax.axis_index("s")
    wid = sid * _NC + cid
    base0 = wid * _EW
    rbase = sid * _RPT

    pltpu.sync_copy(z128_h, hacc.at[pl.ds(rbase, _RPT)])
    pltpu.sync_copy(z16_h, xacc.at[pl.ds(rbase, _RPT)])
    plsc.subcore_barrier()

    def body(i, carry):
        base = base0 + i * _C
        pltpu.sync_copy(row_h.at[pl.ds(base, _C)], idx)
        pltpu.sync_copy(ma_h.at[pl.ds(base, _C)], bufm)
        pltpu.sync_copy(cu_h.at[pl.ds(base, _C)], bufc)
        pltpu.sync_copy(bufm, hacc.at[idx], add=True)
        pltpu.sync_copy(bufc, xacc.at[idx], add=True)
        return carry

    lax.fori_loop(0, _NCHUNK, body, 0)
    plsc.subcore_barrier()

    pltpu.sync_copy(hacc.at[pl.ds(rbase, _RPT)], hp_h.at[cid, pl.ds(rbase, _RPT)])
    pltpu.sync_copy(xacc.at[pl.ds(rbase, _RPT)], xp_h.at[cid, pl.ds(rbase, _RPT)])


def _tc_prologue(hpad, t11, neT, neb, tw1T, tb1, tw2T, tb2, whrT, whcT):
    def body(t_ref, h_ref, neT_ref, neb_ref, tw1T_ref, tb1_ref, tw2T_ref,
             tb2_ref, whrT_ref, whcT_ref, h0_ref, hr_ref, hc_ref):
        tval = t_ref[0, 0]
        half = _TDIM // 2
        lane_i = lax.broadcasted_iota(jnp.int32, (1, _TDIM), 1)
        lane = lane_i.astype(jnp.float32)
        k = jnp.where(lane < half, lane, lane - half)
        freq = jnp.exp(k * (-(np.log(10000.0) / (half - 1))))
        arg = tval * freq
        te0 = jnp.where(lane < half, jnp.sin(arg), jnp.cos(arg))
        te1 = _silu(jnp.dot(te0, tw1T_ref[...], preferred_element_type=jnp.float32)
                    + tb1_ref[...])
        te2 = (jnp.dot(te1, tw2T_ref[...], preferred_element_type=jnp.float32)
               + tb2_ref[...])
        h0 = (jnp.dot(h_ref[...], neT_ref[...], preferred_element_type=jnp.float32)
              + neb_ref[...] + te2)
        h0_ref[...] = h0
        hr_ref[...] = jnp.dot(h0, whrT_ref[...],
                              preferred_element_type=jnp.float32).astype(jnp.bfloat16)
        hc_ref[...] = jnp.dot(h0, whcT_ref[...],
                              preferred_element_type=jnp.float32).astype(jnp.bfloat16)

    full = lambda shape: pl.BlockSpec(shape, lambda i: (0, 0))
    return pl.pallas_call(
        body,
        grid=(_NBN,),
        in_specs=[
            pl.BlockSpec((1, 1), lambda i: (0, 0), memory_space=pltpu.SMEM),
            pl.BlockSpec((_BN, _HID), lambda i: (i, 0)),
            full((_HID, _HID)), full((1, _HID)),
            full((_TDIM, _HID)), full((1, _HID)),
            full((_HID, _HID)), full((1, _HID)),
            full((_HID, _HID)), full((_HID, _HID)),
        ],
        out_specs=[
            pl.BlockSpec((_BN, _HID), lambda i: (i, 0)),
            pl.BlockSpec((_BN, _HID), lambda i: (i, 0)),
            pl.BlockSpec((_BN, _HID), lambda i: (i, 0)),
        ],
        out_shape=[
            jax.ShapeDtypeStruct((_NPAD, _HID), jnp.float32),
            jax.ShapeDtypeStruct((_NPAD, _HID), jnp.bfloat16),
            jax.ShapeDtypeStruct((_NPAD, _HID), jnp.bfloat16),
        ],
        name="egnn_tc_prologue",
    )(t11, hpad, neT, neb, tw1T, tb1, tw2T, tb2, whrT, whcT)


def _tc_edge(gr, gc, xr, xc, ea, w_r, w_eaT, eb1, ew2T, eb2, aw, ab11,
             cw1T, cb1, cw2):
    def body(ab_ref, gr_ref, gc_ref, xr_ref, xc_ref, ea_ref, wr_ref, weaT_ref,
             eb1_ref, ew2T_ref, eb2_ref, aw_ref, cw1T_ref, cb1_ref, cw2_ref,
             ma_ref, cu_ref):
        g = (gr_ref[...] + gc_ref[...]).astype(jnp.float32)
        xd = xr_ref[...] - xc_ref[...]
        radial = jnp.sum(xd * xd, axis=-1, keepdims=True)
        pre = (g + radial * wr_ref[...]
               + jnp.dot(ea_ref[...], weaT_ref[...],
                         preferred_element_type=jnp.float32)
               + eb1_ref[...])
        m = _silu(pre)
        m = _silu(jnp.dot(m, ew2T_ref[...], preferred_element_type=jnp.float32)
                  + eb2_ref[...])
        att = _sigmoid(jnp.sum(m * aw_ref[...], axis=-1, keepdims=True)
                       + ab_ref[0, 0])
        m = m * att
        c1 = _silu(jnp.dot(m, cw1T_ref[...], preferred_element_type=jnp.float32)
                   + cb1_ref[...])
        cws = jnp.sum(c1 * cw2_ref[...], axis=-1, keepdims=True)
        cu_ref[...] = xd * (cws / jnp.sqrt(radial + 1e-08))
        ma_ref[...] = m

    full = lambda shape: pl.BlockSpec(shape, lambda i: (0, 0))
    eb = lambda w: pl.BlockSpec((_BE, w), lambda i: (i, 0))
    return pl.pallas_call(
        body,
        grid=(_E // _BE,),
        in_specs=[
            pl.BlockSpec((1, 1), lambda i: (0, 0), memory_space=pltpu.SMEM),
            eb(_HID), eb(_HID), eb(_XW), eb(_XW), eb(_EDIM),
            full((1, _HID)), full((_EDIM, _HID)), full((1, _HID)),
            full((_HID, _HID)), full((1, _HID)), full((1, _HID)),
            full((_HID, _HID)), full((1, _HID)), full((1, _HID)),
        ],
        out_specs=[eb(_HID), eb(_XW)],
        out_shape=[
            jax.ShapeDtypeStruct((_E, _HID), jnp.float32),
            jax.ShapeDtypeStruct((_E, _XW), jnp.float32),
        ],
        name="egnn_tc_edge",
    )(ab11, gr, gc, xr, xc, ea, w_r, w_eaT, eb1, ew2T, eb2, aw, cw1T, cb1, cw2)


def _tc_node(h, hp0, hp1, nw1hT, nw1mT, nb1, nw2T, nb2, ln_g, ln_b,
             whrT, whcT):
    def body(h_ref, hp0_ref, hp1_ref, nw1hT_ref, nw1mT_ref, nb1_ref, nw2T_ref,
             nb2_ref, g_ref, b_ref, whrT_ref, whcT_ref,
             hn_ref, hr_ref, hc_ref):
        hv = h_ref[...]
        mi = hp0_ref[...] + hp1_ref[...]
        a = _silu(jnp.dot(hv, nw1hT_ref[...], preferred_element_type=jnp.float32)
                  + jnp.dot(mi, nw1mT_ref[...], preferred_element_type=jnp.float32)
                  + nb1_ref[...])
        hn = hv + jnp.dot(a, nw2T_ref[...], preferred_element_type=jnp.float32) \
            + nb2_ref[...]
        mu = jnp.mean(hn, axis=-1, keepdims=True)
        var = jnp.mean((hn - mu) * (hn - mu), axis=-1, keepdims=True)
        hn = (hn - mu) / jnp.sqrt(var + 1e-05) * g_ref[...] + b_ref[...]
        hn_ref[...] = hn
        hr_ref[...] = jnp.dot(hn, whrT_ref[...],
                              preferred_element_type=jnp.float32).astype(jnp.bfloat16)
        hc_ref[...] = jnp.dot(hn, whcT_ref[...],
                              preferred_element_type=jnp.float32).astype(jnp.bfloat16)

    full = lambda shape: pl.BlockSpec(shape, lambda i: (0, 0))
    nb = pl.BlockSpec((_BN, _HID), lambda i: (i, 0))
    return pl.pallas_call(
        body,
        grid=(_NBN,),
        in_specs=[
            nb, nb, nb,
            full((_HID, _HID)), full((_HID, _HID)), full((1, _HID)),
            full((_HID, _HID)), full((1, _HID)), full((1, _HID)),
            full((1, _HID)), full((_HID, _HID)), full((_HID, _HID)),
        ],
        out_specs=[nb, nb, nb],
        out_shape=[
            jax.ShapeDtypeStruct((_NPAD, _HID), jnp.float32),
            jax.ShapeDtypeStruct((_NPAD, _HID), jnp.bfloat16),
            jax.ShapeDtypeStruct((_NPAD, _HID), jnp.bfloat16),
        ],
        name="egnn_tc_node",
    )(h, hp0, hp1, nw1hT, nw1mT, nb1, nw2T, nb2, ln_g, ln_b, whrT, whcT)


def _tc_epilogue(h, om1T, omb1, om2T, omb2, chT16, chb16):
    def body(h_ref, om1T_ref, omb1_ref, om2T_ref, omb2_ref, chT_ref, chb_ref,
             ho_ref, xd_ref):
        hv = h_ref[...]
        a = _silu(jnp.dot(hv, om1T_ref[...], preferred_element_type=jnp.float32)
                  + omb1_ref[...])
        ho_ref[...] = jnp.dot(a, om2T_ref[...], preferred_element_type=jnp.float32) \
            + omb2_ref[...]
        xd_ref[...] = jnp.dot(hv, chT_ref[...], preferred_element_type=jnp.float32) \
            + chb_ref[...]

    full = lambda shape: pl.BlockSpec(shape, lambda i: (0, 0))
    nb = pl.BlockSpec((_BN, _HID), lambda i: (i, 0))
    return pl.pallas_call(
        body,
        grid=(_NBN,),
        in_specs=[
            nb,
            full((_HID, _HID)), full((1, _HID)),
            full((_HID, _HID)), full((1, _HID)),
            full((_HID, _XW)), full((1, _XW)),
        ],
        out_specs=[nb, pl.BlockSpec((_BN, _XW), lambda i: (i, 0))],
        out_shape=[
            jax.ShapeDtypeStruct((_NPAD, _HID), jnp.float32),
            jax.ShapeDtypeStruct((_NPAD, _XW), jnp.float32),
        ],
        name="egnn_tc_epilogue",
    )(h, om1T, omb1, om2T, omb2, chT16, chb16)


def kernel(h, x, edge_index, t, edge_attr, params):
    p = params
    row = edge_index[0]
    col = edge_index[1]

    hpad = jnp.zeros((_NPAD, _HID), jnp.float32).at[:_N].set(h)
    xpad = jnp.zeros((_NPAD, _XW), jnp.float32).at[:_N, :3].set(x)
    x_init = xpad
    t11 = t.reshape(1, 1)
    z128 = jnp.zeros((_RPT, _HID), jnp.float32)
    z16 = jnp.zeros((_RPT, _XW), jnp.float32)

    whrT = [p['ew1'][i][:, :_HID].T for i in range(_L)]
    whcT = [p['ew1'][i][:, _HID:2 * _HID].T for i in range(_L)]
    w_r = [p['ew1'][i][:, 2 * _HID].reshape(1, _HID) for i in range(_L)]
    w_eaT = [p['ew1'][i][:, 2 * _HID + 1:].T for i in range(_L)]

    hcur, hrp, hcp = _tc_prologue(
        hpad, t11,
        p['ne_w'].T, p['ne_b'].reshape(1, _HID),
        p['te_w1'].T, p['te_b1'].reshape(1, _HID),
        p['te_w2'].T, p['te_b2'].reshape(1, _HID),
        whrT[0], whcT[0])

    xcur = xpad
    for i in range(_L):
        gr, gc, xr, xc = _sc_gather(hrp, hcp, xcur, row, col)
        ma, cu = _tc_edge(
            gr, gc, xr, xc, edge_attr,
            w_r[i], w_eaT[i], p['eb1'][i].reshape(1, _HID),
            p['ew2'][i].T, p['eb2'][i].reshape(1, _HID),
            p['aw'][i], p['ab'][i].reshape(1, 1),
            p['cw1'][i].T, p['cb1'][i].reshape(1, _HID), p['cw2'][i])
        hp, xp = _sc_scatter(ma, cu, row, z128, z16)
        j = min(i + 1, _L - 1)
        hcur, hrp, hcp = _tc_node(
            hcur, hp[0], hp[1],
            p['nw1'][i][:, :_HID].T, p['nw1'][i][:, _HID:].T,
            p['nb1'][i].reshape(1, _HID),
            p['nw2'][i].T, p['nb2'][i].reshape(1, _HID),
            p['ln_g'][i].reshape(1, _HID), p['ln_b'][i].reshape(1, _HID),
            whrT[j], whcT[j])
        xcur = xcur + xp[0] + xp[1]

    chT16 = jnp.zeros((_HID, _XW), jnp.float32).at[:, :3].set(p['ch_w'].T)
    chb16 = jnp.zeros((1, _XW), jnp.float32).at[0, :3].set(p['ch_b'])
    hout, xd = _tc_epilogue(
        hcur,
        p['om_w1'].T, p['om_b1'].reshape(1, _HID),
        p['om_w2'].T, p['om_b2'].reshape(1, _HID),
        chT16, chb16)

    x_out = (xcur - x_init)[:_N, :3] + xd[:_N, :3]
    return (hout[:_N], x_out)


# speedup vs baseline: 1.7634x; 1.7634x over previous
import functools

import numpy as np
import jax
import jax.numpy as jnp
from jax import lax
from jax.experimental import pallas as pl
from jax.experimental.pallas import tpu as pltpu
from jax.experimental.pallas import tpu_sc as plsc

_N = 10000
_NPAD = 10240
_E = 320000
_HID = 128
_EDIM = 16
_TDIM = 64
_L = 4
_XW = 16

_NC = 2
_NS = 16
_NW = _NC * _NS

_K = 2
_EP = _E // _K
_EW = _EP // _NW
_C = 200
_NCHUNK = _EW // _C
_RPT = _NPAD // _NS

_BE = 2000
_BN = 512
_NBN = _NPAD // _BN


def _silu(v):
    return v * (1.0 / (1.0 + jnp.exp(-v)))


def _sigmoid(v):
    return 1.0 / (1.0 + jnp.exp(-v))


@functools.partial(
    pl.kernel,
    out_type=(
        jax.ShapeDtypeStruct((_EP, _HID), jnp.float32),
        jax.ShapeDtypeStruct((_EP, _HID), jnp.float32),
        jax.ShapeDtypeStruct((_EP, _XW), jnp.float32),
        jax.ShapeDtypeStruct((_EP, _XW), jnp.float32),
    ),
    mesh=plsc.VectorSubcoreMesh(core_axis_name="c", subcore_axis_name="s"),
    scratch_types=[
        pltpu.VMEM((_C,), jnp.int32),
        pltpu.VMEM((_C,), jnp.int32),
        pltpu.VMEM((_C, _HID), jnp.float32),
        pltpu.VMEM((_C, _HID), jnp.float32),
        pltpu.VMEM((_C, _XW), jnp.float32),
        pltpu.VMEM((_C, _XW), jnp.float32),
        pltpu.SemaphoreType.DMA,
        pltpu.SemaphoreType.DMA,
        pltpu.SemaphoreType.DMA,
        pltpu.SemaphoreType.DMA,
    ],
    compiler_params=pltpu.CompilerParams(use_tc_tiling_on_sc=False),
    name="egnn_sc_gather",
)
def _sc_gather(hrp_h, hcp_h, xp_h, row_h, col_h, gr_h, gc_h, xr_h, xc_h,
               idxr, idxc, bufr, bufc, bufxr, bufxc, s1, s2, s3, s4):
    wid = lax.axis_index("s") * _NC + lax.axis_index("c")
    base0 = wid * _EW

    def body(i, carry):
        base = base0 + i * _C
        pltpu.sync_copy(row_h.at[pl.ds(base, _C)], idxr)
        pltpu.sync_copy(col_h.at[pl.ds(base, _C)], idxc)
        c1 = pltpu.async_copy(hrp_h.at[idxr], bufr, s1)
        c2 = pltpu.async_copy(hcp_h.at[idxc], bufc, s2)
        c3 = pltpu.async_copy(xp_h.at[idxr], bufxr, s3)
        c4 = pltpu.async_copy(xp_h.at[idxc], bufxc, s4)
        c1.wait()
        c2.wait()
        c3.wait()
        c4.wait()
        pltpu.sync_copy(bufr, gr_h.at[pl.ds(base, _C)])
        pltpu.sync_copy(bufc, gc_h.at[pl.ds(base, _C)])
        pltpu.sync_copy(bufxr, xr_h.at[pl.ds(base, _C)])
        pltpu.sync_copy(bufxc, xc_h.at[pl.ds(base, _C)])
        return carry

    lax.fori_loop(0, _NCHUNK, body, 0)


@functools.partial(
    pl.kernel,
    out_type=(
        jax.ShapeDtypeStruct((_NC, _NPAD, _HID), jnp.float32),
        jax.ShapeDtypeStruct((_NC, _NPAD, _XW), jnp.float32),
    ),
    mesh=plsc.VectorSubcoreMesh(core_axis_name="c", subcore_axis_name="s"),
    scratch_types=[
        pltpu.VMEM((_C,), jnp.int32),
        pltpu.VMEM((_C, _HID), jnp.float32),
        pltpu.VMEM((_C, _XW), jnp.float32),
        pltpu.VMEM_SHARED((_NPAD, _HID), jnp.float32),
        pltpu.VMEM_SHARED((_NPAD, _XW), jnp.float32),
    ],
    compiler_params=pltpu.CompilerParams(use_tc_tiling_on_sc=False),
    name="egnn_sc_scatter",
)
def _sc_scatter(ma_h, cu_h, row_h, z128_h, z16_h, hp_h, xp_h,
                idx, bufm, bufc, hacc, xacc):
    cid = lax.axis_index("c")
    sid = lax.axis_index("s")
    wid = sid * _NC + cid
    base0 = wid * _EW
    rbase = sid * _RPT

    pltpu.sync_copy(z128_h, hacc.at[pl.ds(rbase, _RPT)])
    pltpu.sync_copy(z16_h, xacc.at[pl.ds(rbase, _RPT)])
    plsc.subcore_barrier()

    def body(i, carry):
        base = base0 + i * _C
        pltpu.sync_copy(row_h.at[pl.ds(base, _C)], idx)
        pltpu.sync_copy(ma_h.at[pl.ds(base, _C)], bufm)
        pltpu.sync_copy(cu_h.at[pl.ds(base, _C)], bufc)
        pltpu.sync_copy(bufm, hacc.at[idx], add=True)
        pltpu.sync_copy(bufc, xacc.at[idx], add=True)
        return carry

    lax.fori_loop(0, _NCHUNK, body, 0)
    plsc.subcore_barrier()

    pltpu.sync_copy(hacc.at[pl.ds(rbase, _RPT)], hp_h.at[cid, pl.ds(rbase, _RPT)])
    pltpu.sync_copy(xacc.at[pl.ds(rbase, _RPT)], xp_h.at[cid, pl.ds(rbase, _RPT)])


def _tc_prologue(hpad, t11, neT, neb, tw1T, tb1, tw2T, tb2, whrT, whcT):
    def body(t_ref, h_ref, neT_ref, neb_ref, tw1T_ref, tb1_ref, tw2T_ref,
             tb2_ref, whrT_ref, whcT_ref, h0_ref, hr_ref, hc_ref):
        tval = t_ref[0, 0]
        half = _TDIM // 2
        lane_i = lax.broadcasted_iota(jnp.int32, (1, _TDIM), 1)
        lane = lane_i.astype(jnp.float32)
        k = jnp.where(lane < half, lane, lane - half)
        freq = jnp.exp(k * (-(np.log(10000.0) / (half - 1))))
        arg = tval * freq
        te0 = jnp.where(lane < half, jnp.sin(arg), jnp.cos(arg))
        te1 = _silu(jnp.dot(te0, tw1T_ref[...], preferred_element_type=jnp.float32)
                    + tb1_ref[...])
        te2 = (jnp.dot(te1, tw2T_ref[...], preferred_element_type=jnp.float32)
               + tb2_ref[...])
        h0 = (jnp.dot(h_ref[...], neT_ref[...], preferred_element_type=jnp.float32)
              + neb_ref[...] + te2)
        h0_ref[...] = h0
        hr_ref[...] = jnp.dot(h0, whrT_ref[...], preferred_element_type=jnp.float32)
        hc_ref[...] = jnp.dot(h0, whcT_ref[...], preferred_element_type=jnp.float32)

    full = lambda shape: pl.BlockSpec(shape, lambda i: (0, 0))
    return pl.pallas_call(
        body,
        grid=(_NBN,),
        in_specs=[
            pl.BlockSpec((1, 1), lambda i: (0, 0), memory_space=pltpu.SMEM),
            pl.BlockSpec((_BN, _HID), lambda i: (i, 0)),
            full((_HID, _HID)), full((1, _HID)),
            full((_TDIM, _HID)), full((1, _HID)),
            full((_HID, _HID)), full((1, _HID)),
            full((_HID, _HID)), full((_HID, _HID)),
        ],
        out_specs=[
            pl.BlockSpec((_BN, _HID), lambda i: (i, 0)),
            pl.BlockSpec((_BN, _HID), lambda i: (i, 0)),
            pl.BlockSpec((_BN, _HID), lambda i: (i, 0)),
        ],
        out_shape=[
            jax.ShapeDtypeStruct((_NPAD, _HID), jnp.float32),
            jax.ShapeDtypeStruct((_NPAD, _HID), jnp.float32),
            jax.ShapeDtypeStruct((_NPAD, _HID), jnp.float32),
        ],
        name="egnn_tc_prologue",
    )(t11, hpad, neT, neb, tw1T, tb1, tw2T, tb2, whrT, whcT)


def _tc_edge(gr, gc, xr, xc, ea, w_r, w_eaT, eb1, ew2T, eb2, aw, ab11,
             cw1T, cb1, cw2):
    def body(ab_ref, gr_ref, gc_ref, xr_ref, xc_ref, ea_ref, wr_ref, weaT_ref,
             eb1_ref, ew2T_ref, eb2_ref, aw_ref, cw1T_ref, cb1_ref, cw2_ref,
             ma_ref, cu_ref):
        g = gr_ref[...] + gc_ref[...]
        xd = xr_ref[...] - xc_ref[...]
        radial = jnp.sum(xd * xd, axis=-1, keepdims=True)
        pre = (g + radial * wr_ref[...]
               + jnp.dot(ea_ref[...], weaT_ref[...],
                         preferred_element_type=jnp.float32)
               + eb1_ref[...])
        m = _silu(pre)
        m = _silu(jnp.dot(m, ew2T_ref[...], preferred_element_type=jnp.float32)
                  + eb2_ref[...])
        att = _sigmoid(jnp.sum(m * aw_ref[...], axis=-1, keepdims=True)
                       + ab_ref[0, 0])
        m = m * att
        c1 = _silu(jnp.dot(m, cw1T_ref[...], preferred_element_type=jnp.float32)
                   + cb1_ref[...])
        cws = jnp.sum(c1 * cw2_ref[...], axis=-1, keepdims=True)
        cu_ref[...] = xd * (cws / jnp.sqrt(radial + 1e-08))
        ma_ref[...] = m

    full = lambda shape: pl.BlockSpec(shape, lambda i: (0, 0))
    eb = lambda w: pl.BlockSpec((_BE, w), lambda i: (i, 0))
    return pl.pallas_call(
        body,
        grid=(_EP // _BE,),
        in_specs=[
            pl.BlockSpec((1, 1), lambda i: (0, 0), memory_space=pltpu.SMEM),
            eb(_HID), eb(_HID), eb(_XW), eb(_XW), eb(_EDIM),
            full((1, _HID)), full((_EDIM, _HID)), full((1, _HID)),
            full((_HID, _HID)), full((1, _HID)), full((1, _HID)),
            full((_HID, _HID)), full((1, _HID)), full((1, _HID)),
        ],
        out_specs=[eb(_HID), eb(_XW)],
        out_shape=[
            jax.ShapeDtypeStruct((_EP, _HID), jnp.float32),
            jax.ShapeDtypeStruct((_EP, _XW), jnp.float32),
        ],
        name="egnn_tc_edge",
    )(ab11, gr, gc, xr, xc, ea, w_r, w_eaT, eb1, ew2T, eb2, aw, cw1T, cb1, cw2)


def _tc_node(h, hps, nw1hT, nw1mT, nb1, nw2T, nb2, ln_g, ln_b,
             whrT, whcT):
    nparts = len(hps)

    def body(*refs):
        h_ref = refs[0]
        hp_refs = refs[1:1 + nparts]
        (nw1hT_ref, nw1mT_ref, nb1_ref, nw2T_ref, nb2_ref, g_ref, b_ref,
         whrT_ref, whcT_ref, hn_ref, hr_ref, hc_ref) = refs[1 + nparts:]
        hv = h_ref[...]
        mi = hp_refs[0][...]
        for r in hp_refs[1:]:
            mi = mi + r[...]
        a = _silu(jnp.dot(hv, nw1hT_ref[...], preferred_element_type=jnp.float32)
                  + jnp.dot(mi, nw1mT_ref[...], preferred_element_type=jnp.float32)
                  + nb1_ref[...])
        hn = hv + jnp.dot(a, nw2T_ref[...], preferred_element_type=jnp.float32) \
            + nb2_ref[...]
        mu = jnp.mean(hn, axis=-1, keepdims=True)
        var = jnp.mean((hn - mu) * (hn - mu), axis=-1, keepdims=True)
        hn = (hn - mu) / jnp.sqrt(var + 1e-05) * g_ref[...] + b_ref[...]
        hn_ref[...] = hn
        hr_ref[...] = jnp.dot(hn, whrT_ref[...], preferred_element_type=jnp.float32)
        hc_ref[...] = jnp.dot(hn, whcT_ref[...], preferred_element_type=jnp.float32)

    full = lambda shape: pl.BlockSpec(shape, lambda i: (0, 0))
    nb = pl.BlockSpec((_BN, _HID), lambda i: (i, 0))
    return pl.pallas_call(
        body,
        grid=(_NBN,),
        in_specs=[nb] * (1 + nparts) + [
            full((_HID, _HID)), full((_HID, _HID)), full((1, _HID)),
            full((_HID, _HID)), full((1, _HID)), full((1, _HID)),
            full((1, _HID)), full((_HID, _HID)), full((_HID, _HID)),
        ],
        out_specs=[nb, nb, nb],
        out_shape=[
            jax.ShapeDtypeStruct((_NPAD, _HID), jnp.float32),
            jax.ShapeDtypeStruct((_NPAD, _HID), jnp.float32),
            jax.ShapeDtypeStruct((_NPAD, _HID), jnp.float32),
        ],
        name="egnn_tc_node",
    )(h, *hps, nw1hT, nw1mT, nb1, nw2T, nb2, ln_g, ln_b, whrT, whcT)


def _tc_epilogue(h, om1T, omb1, om2T, omb2, chT16, chb16):
    def body(h_ref, om1T_ref, omb1_ref, om2T_ref, omb2_ref, chT_ref, chb_ref,
             ho_ref, xd_ref):
        hv = h_ref[...]
        a = _silu(jnp.dot(hv, om1T_ref[...], preferred_element_type=jnp.float32)
                  + omb1_ref[...])
        ho_ref[...] = jnp.dot(a, om2T_ref[...], preferred_element_type=jnp.float32) \
            + omb2_ref[...]
        xd_ref[...] = jnp.dot(hv, chT_ref[...], preferred_element_type=jnp.float32) \
            + chb_ref[...]

    full = lambda shape: pl.BlockSpec(shape, lambda i: (0, 0))
    nb = pl.BlockSpec((_BN, _HID), lambda i: (i, 0))
    return pl.pallas_call(
        body,
        grid=(_NBN,),
        in_specs=[
            nb,
            full((_HID, _HID)), full((1, _HID)),
            full((_HID, _HID)), full((1, _HID)),
            full((_HID, _XW)), full((1, _XW)),
        ],
        out_specs=[nb, pl.BlockSpec((_BN, _XW), lambda i: (i, 0))],
        out_shape=[
            jax.ShapeDtypeStruct((_NPAD, _HID), jnp.float32),
            jax.ShapeDtypeStruct((_NPAD, _XW), jnp.float32),
        ],
        name="egnn_tc_epilogue",
    )(h, om1T, omb1, om2T, omb2, chT16, chb16)


def kernel(h, x, edge_index, t, edge_attr, params):
    p = params
    rows = [edge_index[0, k * _EP:(k + 1) * _EP] for k in range(_K)]
    cols = [edge_index[1, k * _EP:(k + 1) * _EP] for k in range(_K)]
    eas = [edge_attr[k * _EP:(k + 1) * _EP] for k in range(_K)]

    hpad = jnp.zeros((_NPAD, _HID), jnp.float32).at[:_N].set(h)
    xpad = jnp.zeros((_NPAD, _XW), jnp.float32).at[:_N, :3].set(x)
    x_init = xpad
    t11 = t.reshape(1, 1)
    z128 = jnp.zeros((_RPT, _HID), jnp.float32)
    z16 = jnp.zeros((_RPT, _XW), jnp.float32)

    whrT = [p['ew1'][i][:, :_HID].T for i in range(_L)]
    whcT = [p['ew1'][i][:, _HID:2 * _HID].T for i in range(_L)]
    w_r = [p['ew1'][i][:, 2 * _HID].reshape(1, _HID) for i in range(_L)]
    w_eaT = [p['ew1'][i][:, 2 * _HID + 1:].T for i in range(_L)]

    hcur, hrp, hcp = _tc_prologue(
        hpad, t11,
        p['ne_w'].T, p['ne_b'].reshape(1, _HID),
        p['te_w1'].T, p['te_b1'].reshape(1, _HID),
        p['te_w2'].T, p['te_b2'].reshape(1, _HID),
        whrT[0], whcT[0])

    xcur = xpad
    for i in range(_L):
        hparts = []
        xparts = []
        for k in range(_K):
            gr, gc, xr, xc = _sc_gather(hrp, hcp, xcur, rows[k], cols[k])
            ma, cu = _tc_edge(
                gr, gc, xr, xc, eas[k],
                w_r[i], w_eaT[i], p['eb1'][i].reshape(1, _HID),
                p['ew2'][i].T, p['eb2'][i].reshape(1, _HID),
                p['aw'][i], p['ab'][i].reshape(1, 1),
                p['cw1'][i].T, p['cb1'][i].reshape(1, _HID), p['cw2'][i])
            hp, xp = _sc_scatter(ma, cu, rows[k], z128, z16)
            hparts.extend([hp[0], hp[1]])
            xparts.extend([xp[0], xp[1]])
        j = min(i + 1, _L - 1)
        hcur, hrp, hcp = _tc_node(
            hcur, hparts,
            p['nw1'][i][:, :_HID].T, p['nw1'][i][:, _HID:].T,
            p['nb1'][i].reshape(1, _HID),
            p['nw2'][i].T, p['nb2'][i].reshape(1, _HID),
            p['ln_g'][i].reshape(1, _HID), p['ln_b'][i].reshape(1, _HID),
            whrT[j], whcT[j])
        for xp_part in xparts:
            xcur = xcur + xp_part

    chT16 = jnp.zeros((_HID, _XW), jnp.float32).at[:, :3].set(p['ch_w'].T)
    chb16 = jnp.zeros((1, _XW), jnp.float32).at[0, :3].set(p['ch_b'])
    hout, xd = _tc_epilogue(
        hcur,
        p['om_w1'].T, p['om_b1'].reshape(1, _HID),
        p['om_w2'].T, p['om_b2'].reshape(1, _HID),
        chT16, chb16)

    x_out = (xcur - x_init)[:_N, :3] + xd[:_N, :3]
    return (hout[:_N], x_out)


# speedup vs baseline: 1.7728x; 1.0054x over previous
import functools

import numpy as np
import jax
import jax.numpy as jnp
from jax import lax
from jax.experimental import pallas as pl
from jax.experimental.pallas import tpu as pltpu
from jax.experimental.pallas import tpu_sc as plsc

_N = 10000
_NPAD = 10240
_E = 320000
_HID = 128
_EDIM = 16
_TDIM = 64
_L = 4
_XW = 16

_NC = 2
_NS = 16
_NW = _NC * _NS

_K = 5
_EP = _E // _K
_EW = _EP // _NW
_C = 200
_NCHUNK = _EW // _C
_RPT = _NPAD // _NS

_BE = 2000
_BN = 512
_NBN = _NPAD // _BN


def _silu(v):
    return v * (1.0 / (1.0 + jnp.exp(-v)))


def _sigmoid(v):
    return 1.0 / (1.0 + jnp.exp(-v))


@functools.partial(
    pl.kernel,
    out_type=(
        jax.ShapeDtypeStruct((_EP, _HID), jnp.float32),
        jax.ShapeDtypeStruct((_EP, _HID), jnp.float32),
        jax.ShapeDtypeStruct((_EP, _XW), jnp.float32),
        jax.ShapeDtypeStruct((_EP, _XW), jnp.float32),
    ),
    mesh=plsc.VectorSubcoreMesh(core_axis_name="c", subcore_axis_name="s"),
    scratch_types=[
        pltpu.VMEM((_C,), jnp.int32),
        pltpu.VMEM((_C,), jnp.int32),
        pltpu.VMEM((_C, _HID), jnp.float32),
        pltpu.VMEM((_C, _HID), jnp.float32),
        pltpu.VMEM((_C, _XW), jnp.float32),
        pltpu.VMEM((_C, _XW), jnp.float32),
        pltpu.SemaphoreType.DMA,
        pltpu.SemaphoreType.DMA,
        pltpu.SemaphoreType.DMA,
        pltpu.SemaphoreType.DMA,
    ],
    compiler_params=pltpu.CompilerParams(use_tc_tiling_on_sc=False),
    name="egnn_sc_gather",
)
def _sc_gather(hrp_h, hcp_h, xp_h, row_h, col_h, gr_h, gc_h, xr_h, xc_h,
               idxr, idxc, bufr, bufc, bufxr, bufxc, s1, s2, s3, s4):
    wid = lax.axis_index("s") * _NC + lax.axis_index("c")
    base0 = wid * _EW

    def body(i, carry):
        base = base0 + i * _C
        pltpu.sync_copy(row_h.at[pl.ds(base, _C)], idxr)
        pltpu.sync_copy(col_h.at[pl.ds(base, _C)], idxc)
        c1 = pltpu.async_copy(hrp_h.at[idxr], bufr, s1)
        c2 = pltpu.async_copy(hcp_h.at[idxc], bufc, s2)
        c3 = pltpu.async_copy(xp_h.at[idxr], bufxr, s3)
        c4 = pltpu.async_copy(xp_h.at[idxc], bufxc, s4)
        c1.wait()
        c2.wait()
        c3.wait()
        c4.wait()
        pltpu.sync_copy(bufr, gr_h.at[pl.ds(base, _C)])
        pltpu.sync_copy(bufc, gc_h.at[pl.ds(base, _C)])
        pltpu.sync_copy(bufxr, xr_h.at[pl.ds(base, _C)])
        pltpu.sync_copy(bufxc, xc_h.at[pl.ds(base, _C)])
        return carry

    lax.fori_loop(0, _NCHUNK, body, 0)


@functools.partial(
    pl.kernel,
    out_type=(
        jax.ShapeDtypeStruct((_NC, _NPAD, _HID), jnp.float32),
        jax.ShapeDtypeStruct((_NC, _NPAD, _XW), jnp.float32),
    ),
    mesh=plsc.VectorSubcoreMesh(core_axis_name="c", subcore_axis_name="s"),
    scratch_types=[
        pltpu.VMEM((_C,), jnp.int32),
        pltpu.VMEM((_C, _HID), jnp.float32),
        pltpu.VMEM((_C, _XW), jnp.float32),
        pltpu.VMEM_SHARED((_NPAD, _HID), jnp.float32),
        pltpu.VMEM_SHARED((_NPAD, _XW), jnp.float32),
    ],
    compiler_params=pltpu.CompilerParams(use_tc_tiling_on_sc=False),
    name="egnn_sc_scatter",
)
def _sc_scatter(ma_h, cu_h, row_h, z128_h, z16_h, hp_h, xp_h,
                idx, bufm, bufc, hacc, xacc):
    cid = lax.axis_index("c")
    sid = lax.axis_index("s")
    wid = sid * _NC + cid
    base0 = wid * _EW
    rbase = sid * _RPT

    pltpu.sync_copy(z128_h, hacc.at[pl.ds(rbase, _RPT)])
    pltpu.sync_copy(z16_h, xacc.at[pl.ds(rbase, _RPT)])
    plsc.subcore_barrier()

    def body(i, carry):
        base = base0 + i * _C
        pltpu.sync_copy(row_h.at[pl.ds(base, _C)], idx)
        pltpu.sync_copy(ma_h.at[pl.ds(base, _C)], bufm)
        pltpu.sync_copy(cu_h.at[pl.ds(base, _C)], bufc)
        pltpu.sync_copy(bufm, hacc.at[idx], add=True)
        pltpu.sync_copy(bufc, xacc.at[idx], add=True)
        return carry

    lax.fori_loop(0, _NCHUNK, body, 0)
    plsc.subcore_barrier()

    pltpu.sync_copy(hacc.at[pl.ds(rbase, _RPT)], hp_h.at[cid, pl.ds(rbase, _RPT)])
    pltpu.sync_copy(xacc.at[pl.ds(rbase, _RPT)], xp_h.at[cid, pl.ds(rbase, _RPT)])


def _tc_prologue(hpad, t11, neT, neb, tw1T, tb1, tw2T, tb2, whrT, whcT):
    def body(t_ref, h_ref, neT_ref, neb_ref, tw1T_ref, tb1_ref, tw2T_ref,
             tb2_ref, whrT_ref, whcT_ref, h0_ref, hr_ref, hc_ref):
        tval = t_ref[0, 0]
        half = _TDIM // 2
        lane_i = lax.broadcasted_iota(jnp.int32, (1, _TDIM), 1)
        lane = lane_i.astype(jnp.float32)
        k = jnp.where(lane < half, lane, lane - half)
        freq = jnp.exp(k * (-(np.log(10000.0) / (half - 1))))
        arg = tval * freq
        te0 = jnp.where(lane < half, jnp.sin(arg), jnp.cos(arg))
        te1 = _silu(jnp.dot(te0, tw1T_ref[...], preferred_element_type=jnp.float32)
                    + tb1_ref[...])
        te2 = (jnp.dot(te1, tw2T_ref[...], preferred_element_type=jnp.float32)
               + tb2_ref[...])
        h0 = (jnp.dot(h_ref[...], neT_ref[...], preferred_element_type=jnp.float32)
              + neb_ref[...] + te2)
        h0_ref[...] = h0
        hr_ref[...] = jnp.dot(h0, whrT_ref[...], preferred_element_type=jnp.float32)
        hc_ref[...] = jnp.dot(h0, whcT_ref[...], preferred_element_type=jnp.float32)

    full = lambda shape: pl.BlockSpec(shape, lambda i: (0, 0))
    return pl.pallas_call(
        body,
        grid=(_NBN,),
        in_specs=[
            pl.BlockSpec((1, 1), lambda i: (0, 0), memory_space=pltpu.SMEM),
            pl.BlockSpec((_BN, _HID), lambda i: (i, 0)),
            full((_HID, _HID)), full((1, _HID)),
            full((_TDIM, _HID)), full((1, _HID)),
            full((_HID, _HID)), full((1, _HID)),
            full((_HID, _HID)), full((_HID, _HID)),
        ],
        out_specs=[
            pl.BlockSpec((_BN, _HID), lambda i: (i, 0)),
            pl.BlockSpec((_BN, _HID), lambda i: (i, 0)),
            pl.BlockSpec((_BN, _HID), lambda i: (i, 0)),
        ],
        out_shape=[
            jax.ShapeDtypeStruct((_NPAD, _HID), jnp.float32),
            jax.ShapeDtypeStruct((_NPAD, _HID), jnp.float32),
            jax.ShapeDtypeStruct((_NPAD, _HID), jnp.float32),
        ],
        name="egnn_tc_prologue",
    )(t11, hpad, neT, neb, tw1T, tb1, tw2T, tb2, whrT, whcT)


def _tc_edge(gr, gc, xr, xc, ea, w_r, w_eaT, eb1, ew2T, eb2, aw, ab11,
             cw1T, cb1, cw2):
    def body(ab_ref, gr_ref, gc_ref, xr_ref, xc_ref, ea_ref, wr_ref, weaT_ref,
             eb1_ref, ew2T_ref, eb2_ref, aw_ref, cw1T_ref, cb1_ref, cw2_ref,
             ma_ref, cu_ref):
        g = gr_ref[...] + gc_ref[...]
        xd = xr_ref[...] - xc_ref[...]
        radial = jnp.sum(xd * xd, axis=-1, keepdims=True)
        pre = (g + radial * wr_ref[...]
               + jnp.dot(ea_ref[...], weaT_ref[...],
                         preferred_element_type=jnp.float32)
               + eb1_ref[...])
        m = _silu(pre)
        m = _silu(jnp.dot(m, ew2T_ref[...], preferred_element_type=jnp.float32)
                  + eb2_ref[...])
        att = _sigmoid(jnp.sum(m * aw_ref[...], axis=-1, keepdims=True)
                       + ab_ref[0, 0])
        m = m * att
        c1 = _silu(jnp.dot(m, cw1T_ref[...], preferred_element_type=jnp.float32)
                   + cb1_ref[...])
        cws = jnp.sum(c1 * cw2_ref[...], axis=-1, keepdims=True)
        cu_ref[...] = xd * (cws / jnp.sqrt(radial + 1e-08))
        ma_ref[...] = m

    full = lambda shape: pl.BlockSpec(shape, lambda i: (0, 0))
    eb = lambda w: pl.BlockSpec((_BE, w), lambda i: (i, 0))
    return pl.pallas_call(
        body,
        grid=(_EP // _BE,),
        in_specs=[
            pl.BlockSpec((1, 1), lambda i: (0, 0), memory_space=pltpu.SMEM),
            eb(_HID), eb(_HID), eb(_XW), eb(_XW), eb(_EDIM),
            full((1, _HID)), full((_EDIM, _HID)), full((1, _HID)),
            full((_HID, _HID)), full((1, _HID)), full((1, _HID)),
            full((_HID, _HID)), full((1, _HID)), full((1, _HID)),
        ],
        out_specs=[eb(_HID), eb(_XW)],
        out_shape=[
            jax.ShapeDtypeStruct((_EP, _HID), jnp.float32),
            jax.ShapeDtypeStruct((_EP, _XW), jnp.float32),
        ],
        name="egnn_tc_edge",
    )(ab11, gr, gc, xr, xc, ea, w_r, w_eaT, eb1, ew2T, eb2, aw, cw1T, cb1, cw2)


def _tc_node(h, hps, nw1hT, nw1mT, nb1, nw2T, nb2, ln_g, ln_b,
             whrT, whcT):
    nparts = len(hps)

    def body(*refs):
        h_ref = refs[0]
        hp_refs = refs[1:1 + nparts]
        (nw1hT_ref, nw1mT_ref, nb1_ref, nw2T_ref, nb2_ref, g_ref, b_ref,
         whrT_ref, whcT_ref, hn_ref, hr_ref, hc_ref) = refs[1 + nparts:]
        hv = h_ref[...]
        mi = hp_refs[0][...]
        for r in hp_refs[1:]:
            mi = mi + r[...]
        a = _silu(jnp.dot(hv, nw1hT_ref[...], preferred_element_type=jnp.float32)
                  + jnp.dot(mi, nw1mT_ref[...], preferred_element_type=jnp.float32)
                  + nb1_ref[...])
        hn = hv + jnp.dot(a, nw2T_ref[...], preferred_element_type=jnp.float32) \
            + nb2_ref[...]
        mu = jnp.mean(hn, axis=-1, keepdims=True)
        var = jnp.mean((hn - mu) * (hn - mu), axis=-1, keepdims=True)
        hn = (hn - mu) / jnp.sqrt(var + 1e-05) * g_ref[...] + b_ref[...]
        hn_ref[...] = hn
        hr_ref[...] = jnp.dot(hn, whrT_ref[...], preferred_element_type=jnp.float32)
        hc_ref[...] = jnp.dot(hn, whcT_ref[...], preferred_element_type=jnp.float32)

    full = lambda shape: pl.BlockSpec(shape, lambda i: (0, 0))
    nb = pl.BlockSpec((_BN, _HID), lambda i: (i, 0))
    return pl.pallas_call(
        body,
        grid=(_NBN,),
        in_specs=[nb] * (1 + nparts) + [
            full((_HID, _HID)), full((_HID, _HID)), full((1, _HID)),
            full((_HID, _HID)), full((1, _HID)), full((1, _HID)),
            full((1, _HID)), full((_HID, _HID)), full((_HID, _HID)),
        ],
        out_specs=[nb, nb, nb],
        out_shape=[
            jax.ShapeDtypeStruct((_NPAD, _HID), jnp.float32),
            jax.ShapeDtypeStruct((_NPAD, _HID), jnp.float32),
            jax.ShapeDtypeStruct((_NPAD, _HID), jnp.float32),
        ],
        name="egnn_tc_node",
    )(h, *hps, nw1hT, nw1mT, nb1, nw2T, nb2, ln_g, ln_b, whrT, whcT)


def _tc_epilogue(h, om1T, omb1, om2T, omb2, chT16, chb16):
    def body(h_ref, om1T_ref, omb1_ref, om2T_ref, omb2_ref, chT_ref, chb_ref,
             ho_ref, xd_ref):
        hv = h_ref[...]
        a = _silu(jnp.dot(hv, om1T_ref[...], preferred_element_type=jnp.float32)
                  + omb1_ref[...])
        ho_ref[...] = jnp.dot(a, om2T_ref[...], preferred_element_type=jnp.float32) \
            + omb2_ref[...]
        xd_ref[...] = jnp.dot(hv, chT_ref[...], preferred_element_type=jnp.float32) \
            + chb_ref[...]

    full = lambda shape: pl.BlockSpec(shape, lambda i: (0, 0))
    nb = pl.BlockSpec((_BN, _HID), lambda i: (i, 0))
    return pl.pallas_call(
        body,
        grid=(_NBN,),
        in_specs=[
            nb,
            full((_HID, _HID)), full((1, _HID)),
            full((_HID, _HID)), full((1, _HID)),
            full((_HID, _XW)), full((1, _XW)),
        ],
        out_specs=[nb, pl.BlockSpec((_BN, _XW), lambda i: (i, 0))],
        out_shape=[
            jax.ShapeDtypeStruct((_NPAD, _HID), jnp.float32),
            jax.ShapeDtypeStruct((_NPAD, _XW), jnp.float32),
        ],
        name="egnn_tc_epilogue",
    )(h, om1T, omb1, om2T, omb2, chT16, chb16)


def kernel(h, x, edge_index, t, edge_attr, params):
    p = params
    rows = [edge_index[0, k * _EP:(k + 1) * _EP] for k in range(_K)]
    cols = [edge_index[1, k * _EP:(k + 1) * _EP] for k in range(_K)]
    eas = [edge_attr[k * _EP:(k + 1) * _EP] for k in range(_K)]

    hpad = jnp.zeros((_NPAD, _HID), jnp.float32).at[:_N].set(h)
    xpad = jnp.zeros((_NPAD, _XW), jnp.float32).at[:_N, :3].set(x)
    x_init = xpad
    t11 = t.reshape(1, 1)
    z128 = jnp.zeros((_RPT, _HID), jnp.float32)
    z16 = jnp.zeros((_RPT, _XW), jnp.float32)

    whrT = [p['ew1'][i][:, :_HID].T for i in range(_L)]
    whcT = [p['ew1'][i][:, _HID:2 * _HID].T for i in range(_L)]
    w_r = [p['ew1'][i][:, 2 * _HID].reshape(1, _HID) for i in range(_L)]
    w_eaT = [p['ew1'][i][:, 2 * _HID + 1:].T for i in range(_L)]

    hcur, hrp, hcp = _tc_prologue(
        hpad, t11,
        p['ne_w'].T, p['ne_b'].reshape(1, _HID),
        p['te_w1'].T, p['te_b1'].reshape(1, _HID),
        p['te_w2'].T, p['te_b2'].reshape(1, _HID),
        whrT[0], whcT[0])

    xcur = xpad
    for i in range(_L):
        hparts = []
        xparts = []
        for k in range(_K):
            gr, gc, xr, xc = _sc_gather(hrp, hcp, xcur, rows[k], cols[k])
            ma, cu = _tc_edge(
                gr, gc, xr, xc, eas[k],
                w_r[i], w_eaT[i], p['eb1'][i].reshape(1, _HID),
                p['ew2'][i].T, p['eb2'][i].reshape(1, _HID),
                p['aw'][i], p['ab'][i].reshape(1, 1),
                p['cw1'][i].T, p['cb1'][i].reshape(1, _HID), p['cw2'][i])
            hp, xp = _sc_scatter(ma, cu, rows[k], z128, z16)
            hparts.extend([hp[0], hp[1]])
            xparts.extend([xp[0], xp[1]])
        j = min(i + 1, _L - 1)
        hcur, hrp, hcp = _tc_node(
            hcur, hparts,
            p['nw1'][i][:, :_HID].T, p['nw1'][i][:, _HID:].T,
            p['nb1'][i].reshape(1, _HID),
            p['nw2'][i].T, p['nb2'][i].reshape(1, _HID),
            p['ln_g'][i].reshape(1, _HID), p['ln_b'][i].reshape(1, _HID),
            whrT[j], whcT[j])
        for xp_part in xparts:
            xcur = xcur + xp_part

    chT16 = jnp.zeros((_HID, _XW), jnp.float32).at[:, :3].set(p['ch_w'].T)
    chb16 = jnp.zeros((1, _XW), jnp.float32).at[0, :3].set(p['ch_b'])
    hout, xd = _tc_epilogue(
        hcur,
        p['om_w1'].T, p['om_b1'].reshape(1, _HID),
        p['om_w2'].T, p['om_b2'].reshape(1, _HID),
        chT16, chb16)

    x_out = (xcur - x_init)[:_N, :3] + xd[:_N, :3]
    return (hout[:_N], x_out)


# speedup vs baseline: 2.1123x; 1.1915x over previous
import functools

import numpy as np
import jax
import jax.numpy as jnp
from jax import lax
from jax.experimental import pallas as pl
from jax.experimental.pallas import tpu as pltpu
from jax.experimental.pallas import tpu_sc as plsc

_N = 10000
_NPAD = 10240
_E = 320000
_HID = 128
_EDIM = 16
_TDIM = 64
_L = 4
_XW = 16

_NC = 2
_NS = 16
_NW = _NC * _NS

_K = 5
_EP = _E // _K
_EW = _EP // _NW
_C = 200
_NCHUNK = _EW // _C
_RPT = _NPAD // _NS

_BE = 3200
_BN = 512
_NBN = _NPAD // _BN


def _silu(v):
    return v * (1.0 / (1.0 + jnp.exp(-v)))


def _sigmoid(v):
    return 1.0 / (1.0 + jnp.exp(-v))


@functools.partial(
    pl.kernel,
    out_type=(
        jax.ShapeDtypeStruct((_EP, _HID), jnp.float32),
        jax.ShapeDtypeStruct((_EP, _HID), jnp.float32),
        jax.ShapeDtypeStruct((_EP, _HID), jnp.float32),
    ),
    mesh=plsc.VectorSubcoreMesh(core_axis_name="c", subcore_axis_name="s"),
    scratch_types=[
        pltpu.VMEM((_C,), jnp.int32),
        pltpu.VMEM((_C,), jnp.int32),
        pltpu.VMEM((_C, _HID), jnp.float32),
        pltpu.VMEM((_C, _HID), jnp.float32),
        pltpu.VMEM((_C, _XW), jnp.float32),
        pltpu.VMEM((_C, _XW), jnp.float32),
        pltpu.SemaphoreType.DMA,
        pltpu.SemaphoreType.DMA,
        pltpu.SemaphoreType.DMA,
        pltpu.SemaphoreType.DMA,
    ],
    compiler_params=pltpu.CompilerParams(use_tc_tiling_on_sc=False),
    name="egnn_sc_gather",
)
def _sc_gather(hrp_h, hcp_h, xp_h, row_h, col_h, gr_h, gc_h, xrc_h,
               idxr, idxc, bufr, bufc, bufxr, bufxc, s1, s2, s3, s4):
    wid = lax.axis_index("s") * _NC + lax.axis_index("c")
    base0 = wid * _EW

    def body(i, carry):
        base = base0 + i * _C
        pltpu.sync_copy(row_h.at[pl.ds(base, _C)], idxr)
        pltpu.sync_copy(col_h.at[pl.ds(base, _C)], idxc)
        c1 = pltpu.async_copy(hrp_h.at[idxr], bufr, s1)
        c2 = pltpu.async_copy(hcp_h.at[idxc], bufc, s2)
        c3 = pltpu.async_copy(xp_h.at[idxr], bufxr, s3)
        c4 = pltpu.async_copy(xp_h.at[idxc], bufxc, s4)
        c1.wait()
        c2.wait()
        c3.wait()
        c4.wait()
        pltpu.sync_copy(bufr, gr_h.at[pl.ds(base, _C)])
        pltpu.sync_copy(bufc, gc_h.at[pl.ds(base, _C)])
        pltpu.sync_copy(bufxr, xrc_h.at[pl.ds(base, _C), pl.ds(0, _XW)])
        pltpu.sync_copy(bufxc, xrc_h.at[pl.ds(base, _C), pl.ds(_XW, _XW)])
        return carry

    lax.fori_loop(0, _NCHUNK, body, 0)


@functools.partial(
    pl.kernel,
    out_type=(
        jax.ShapeDtypeStruct((_NC, _NPAD, _HID), jnp.float32),
        jax.ShapeDtypeStruct((_NC, _NPAD, _XW), jnp.float32),
    ),
    mesh=plsc.VectorSubcoreMesh(core_axis_name="c", subcore_axis_name="s"),
    scratch_types=[
        pltpu.VMEM((_C,), jnp.int32),
        pltpu.VMEM((_C, _HID), jnp.float32),
        pltpu.VMEM((_C, _XW), jnp.float32),
        pltpu.VMEM_SHARED((_NPAD, _HID), jnp.float32),
        pltpu.VMEM_SHARED((_NPAD, _XW), jnp.float32),
    ],
    compiler_params=pltpu.CompilerParams(use_tc_tiling_on_sc=False),
    name="egnn_sc_scatter",
)
def _sc_scatter(ma_h, cu_h, row_h, z128_h, z16_h, hp_h, xp_h,
                idx, bufm, bufc, hacc, xacc):
    cid = lax.axis_index("c")
    sid = lax.axis_index("s")
    wid = sid * _NC + cid
    base0 = wid * _EW
    rbase = sid * _RPT

    pltpu.sync_copy(z128_h, hacc.at[pl.ds(rbase, _RPT)])
    pltpu.sync_copy(z16_h, xacc.at[pl.ds(rbase, _RPT)])
    plsc.subcore_barrier()

    def body(i, carry):
        base = base0 + i * _C
        pltpu.sync_copy(row_h.at[pl.ds(base, _C)], idx)
        pltpu.sync_copy(ma_h.at[pl.ds(base, _C)], bufm)
        pltpu.sync_copy(cu_h.at[pl.ds(base, _C), pl.ds(0, _XW)], bufc)
        pltpu.sync_copy(bufm, hacc.at[idx], add=True)
        pltpu.sync_copy(bufc, xacc.at[idx], add=True)
        return carry

    lax.fori_loop(0, _NCHUNK, body, 0)
    plsc.subcore_barrier()

    pltpu.sync_copy(hacc.at[pl.ds(rbase, _RPT)], hp_h.at[cid, pl.ds(rbase, _RPT)])
    pltpu.sync_copy(xacc.at[pl.ds(rbase, _RPT)], xp_h.at[cid, pl.ds(rbase, _RPT)])


def _tc_prologue(hpad, t11, neT, neb, tw1T, tb1, tw2T, tb2, whrT, whcT):
    def body(t_ref, h_ref, neT_ref, neb_ref, tw1T_ref, tb1_ref, tw2T_ref,
             tb2_ref, whrT_ref, whcT_ref, h0_ref, hr_ref, hc_ref):
        tval = t_ref[0, 0]
        half = _TDIM // 2
        lane_i = lax.broadcasted_iota(jnp.int32, (1, _TDIM), 1)
        lane = lane_i.astype(jnp.float32)
        k = jnp.where(lane < half, lane, lane - half)
        freq = jnp.exp(k * (-(np.log(10000.0) / (half - 1))))
        arg = tval * freq
        te0 = jnp.where(lane < half, jnp.sin(arg), jnp.cos(arg))
        te1 = _silu(jnp.dot(te0, tw1T_ref[...], preferred_element_type=jnp.float32)
                    + tb1_ref[...])
        te2 = (jnp.dot(te1, tw2T_ref[...], preferred_element_type=jnp.float32)
               + tb2_ref[...])
        h0 = (jnp.dot(h_ref[...], neT_ref[...], preferred_element_type=jnp.float32)
              + neb_ref[...] + te2)
        h0_ref[...] = h0
        hr_ref[...] = jnp.dot(h0, whrT_ref[...], preferred_element_type=jnp.float32)
        hc_ref[...] = jnp.dot(h0, whcT_ref[...], preferred_element_type=jnp.float32)

    full = lambda shape: pl.BlockSpec(shape, lambda i: (0, 0))
    return pl.pallas_call(
        body,
        grid=(_NBN,),
        in_specs=[
            pl.BlockSpec((1, 1), lambda i: (0, 0), memory_space=pltpu.SMEM),
            pl.BlockSpec((_BN, _HID), lambda i: (i, 0)),
            full((_HID, _HID)), full((1, _HID)),
            full((_TDIM, _HID)), full((1, _HID)),
            full((_HID, _HID)), full((1, _HID)),
            full((_HID, _HID)), full((_HID, _HID)),
        ],
        out_specs=[
            pl.BlockSpec((_BN, _HID), lambda i: (i, 0)),
            pl.BlockSpec((_BN, _HID), lambda i: (i, 0)),
            pl.BlockSpec((_BN, _HID), lambda i: (i, 0)),
        ],
        out_shape=[
            jax.ShapeDtypeStruct((_NPAD, _HID), jnp.float32),
            jax.ShapeDtypeStruct((_NPAD, _HID), jnp.float32),
            jax.ShapeDtypeStruct((_NPAD, _HID), jnp.float32),
        ],
        name="egnn_tc_prologue",
    )(t11, hpad, neT, neb, tw1T, tb1, tw2T, tb2, whrT, whcT)


def _tc_edge(gr, gc, xrc, ea, dmat, w_r, w_eaT, eb1, ew2T, eb2, aw, ab11,
             cw1T, cb1, cw2):
    def body(ab_ref, gr_ref, gc_ref, xrc_ref, ea_ref, dmat_ref, wr_ref,
             weaT_ref, eb1_ref, ew2T_ref, eb2_ref, aw_ref, cw1T_ref, cb1_ref,
             cw2_ref, ma_ref, cu_ref):
        g = gr_ref[...] + gc_ref[...]
        lane = lax.broadcasted_iota(jnp.int32, (_BE, _HID), 1)
        xm = jnp.where(lane < 2 * _XW, xrc_ref[...], 0.0)
        xd = jnp.dot(xm, dmat_ref[...], preferred_element_type=jnp.float32)
        radial = jnp.sum(xd * xd, axis=-1, keepdims=True)
        pre = (g + radial * wr_ref[...]
               + jnp.dot(ea_ref[...], weaT_ref[...],
                         preferred_element_type=jnp.float32)
               + eb1_ref[...])
        m = _silu(pre)
        m = _silu(jnp.dot(m, ew2T_ref[...], preferred_element_type=jnp.float32)
                  + eb2_ref[...])
        att = _sigmoid(jnp.sum(m * aw_ref[...], axis=-1, keepdims=True)
                       + ab_ref[0, 0])
        m = m * att
        c1 = _silu(jnp.dot(m, cw1T_ref[...], preferred_element_type=jnp.float32)
                   + cb1_ref[...])
        cws = jnp.sum(c1 * cw2_ref[...], axis=-1, keepdims=True)
        cu_ref[...] = xd * (cws / jnp.sqrt(radial + 1e-08))
        ma_ref[...] = m

    full = lambda shape: pl.BlockSpec(shape, lambda i: (0, 0))
    eb = lambda w: pl.BlockSpec((_BE, w), lambda i: (i, 0))
    return pl.pallas_call(
        body,
        grid=(_EP // _BE,),
        in_specs=[
            pl.BlockSpec((1, 1), lambda i: (0, 0), memory_space=pltpu.SMEM),
            eb(_HID), eb(_HID), eb(_HID), eb(_EDIM),
            full((_HID, _HID)),
            full((1, _HID)), full((_EDIM, _HID)), full((1, _HID)),
            full((_HID, _HID)), full((1, _HID)), full((1, _HID)),
            full((_HID, _HID)), full((1, _HID)), full((1, _HID)),
        ],
        out_specs=[eb(_HID), eb(_HID)],
        out_shape=[
            jax.ShapeDtypeStruct((_EP, _HID), jnp.float32),
            jax.ShapeDtypeStruct((_EP, _HID), jnp.float32),
        ],
        name="egnn_tc_edge",
    )(ab11, gr, gc, xrc, ea, dmat, w_r, w_eaT, eb1, ew2T, eb2, aw,
      cw1T, cb1, cw2)


def _tc_node(h, hps, nw1hT, nw1mT, nb1, nw2T, nb2, ln_g, ln_b,
             whrT, whcT):
    nparts = len(hps)

    def body(*refs):
        h_ref = refs[0]
        hp_refs = refs[1:1 + nparts]
        (nw1hT_ref, nw1mT_ref, nb1_ref, nw2T_ref, nb2_ref, g_ref, b_ref,
         whrT_ref, whcT_ref, hn_ref, hr_ref, hc_ref) = refs[1 + nparts:]
        hv = h_ref[...]
        mi = hp_refs[0][...]
        for r in hp_refs[1:]:
            mi = mi + r[...]
        a = _silu(jnp.dot(hv, nw1hT_ref[...], preferred_element_type=jnp.float32)
                  + jnp.dot(mi, nw1mT_ref[...], preferred_element_type=jnp.float32)
                  + nb1_ref[...])
        hn = hv + jnp.dot(a, nw2T_ref[...], preferred_element_type=jnp.float32) \
            + nb2_ref[...]
        mu = jnp.mean(hn, axis=-1, keepdims=True)
        var = jnp.mean((hn - mu) * (hn - mu), axis=-1, keepdims=True)
        hn = (hn - mu) / jnp.sqrt(var + 1e-05) * g_ref[...] + b_ref[...]
        hn_ref[...] = hn
        hr_ref[...] = jnp.dot(hn, whrT_ref[...], preferred_element_type=jnp.float32)
        hc_ref[...] = jnp.dot(hn, whcT_ref[...], preferred_element_type=jnp.float32)

    full = lambda shape: pl.BlockSpec(shape, lambda i: (0, 0))
    nb = pl.BlockSpec((_BN, _HID), lambda i: (i, 0))
    return pl.pallas_call(
        body,
        grid=(_NBN,),
        in_specs=[nb] * (1 + nparts) + [
            full((_HID, _HID)), full((_HID, _HID)), full((1, _HID)),
            full((_HID, _HID)), full((1, _HID)), full((1, _HID)),
            full((1, _HID)), full((_HID, _HID)), full((_HID, _HID)),
        ],
        out_specs=[nb, nb, nb],
        out_shape=[
            jax.ShapeDtypeStruct((_NPAD, _HID), jnp.float32),
            jax.ShapeDtypeStruct((_NPAD, _HID), jnp.float32),
            jax.ShapeDtypeStruct((_NPAD, _HID), jnp.float32),
        ],
        name="egnn_tc_node",
    )(h, *hps, nw1hT, nw1mT, nb1, nw2T, nb2, ln_g, ln_b, whrT, whcT)


def _tc_epilogue(h, om1T, omb1, om2T, omb2, chT16, chb16):
    def body(h_ref, om1T_ref, omb1_ref, om2T_ref, omb2_ref, chT_ref, chb_ref,
             ho_ref, xd_ref):
        hv = h_ref[...]
        a = _silu(jnp.dot(hv, om1T_ref[...], preferred_element_type=jnp.float32)
                  + omb1_ref[...])
        ho_ref[...] = jnp.dot(a, om2T_ref[...], preferred_element_type=jnp.float32) \
            + omb2_ref[...]
        xd_ref[...] = jnp.dot(hv, chT_ref[...], preferred_element_type=jnp.float32) \
            + chb_ref[...]

    full = lambda shape: pl.BlockSpec(shape, lambda i: (0, 0))
    nb = pl.BlockSpec((_BN, _HID), lambda i: (i, 0))
    return pl.pallas_call(
        body,
        grid=(_NBN,),
        in_specs=[
            nb,
            full((_HID, _HID)), full((1, _HID)),
            full((_HID, _HID)), full((1, _HID)),
            full((_HID, _XW)), full((1, _XW)),
        ],
        out_specs=[nb, pl.BlockSpec((_BN, _XW), lambda i: (i, 0))],
        out_shape=[
            jax.ShapeDtypeStruct((_NPAD, _HID), jnp.float32),
            jax.ShapeDtypeStruct((_NPAD, _XW), jnp.float32),
        ],
        name="egnn_tc_epilogue",
    )(h, om1T, omb1, om2T, omb2, chT16, chb16)


def kernel(h, x, edge_index, t, edge_attr, params):
    p = params
    rows = [edge_index[0, k * _EP:(k + 1) * _EP] for k in range(_K)]
    cols = [edge_index[1, k * _EP:(k + 1) * _EP] for k in range(_K)]
    eas = [edge_attr[k * _EP:(k + 1) * _EP] for k in range(_K)]

    hpad = jnp.zeros((_NPAD, _HID), jnp.float32).at[:_N].set(h)
    xpad = jnp.zeros((_NPAD, _XW), jnp.float32).at[:_N, :3].set(x)
    x_init = xpad
    t11 = t.reshape(1, 1)
    z128 = jnp.zeros((_RPT, _HID), jnp.float32)
    z16 = jnp.zeros((_RPT, _XW), jnp.float32)

    dmat_np = np.zeros((_HID, _HID), np.float32)
    for l in range(_XW):
        dmat_np[l, l] = 1.0
        dmat_np[l + _XW, l] = -1.0
    dmat = jnp.asarray(dmat_np)

    whrT = [p['ew1'][i][:, :_HID].T for i in range(_L)]
    whcT = [p['ew1'][i][:, _HID:2 * _HID].T for i in range(_L)]
    w_r = [p['ew1'][i][:, 2 * _HID].reshape(1, _HID) for i in range(_L)]
    w_eaT = [p['ew1'][i][:, 2 * _HID + 1:].T for i in range(_L)]

    hcur, hrp, hcp = _tc_prologue(
        hpad, t11,
        p['ne_w'].T, p['ne_b'].reshape(1, _HID),
        p['te_w1'].T, p['te_b1'].reshape(1, _HID),
        p['te_w2'].T, p['te_b2'].reshape(1, _HID),
        whrT[0], whcT[0])

    xcur = xpad
    for i in range(_L):
        hparts = []
        xparts = []
        for k in range(_K):
            gr, gc, xrc = _sc_gather(hrp, hcp, xcur, rows[k], cols[k])
            ma, cu = _tc_edge(
                gr, gc, xrc, eas[k], dmat,
                w_r[i], w_eaT[i], p['eb1'][i].reshape(1, _HID),
                p['ew2'][i].T, p['eb2'][i].reshape(1, _HID),
                p['aw'][i], p['ab'][i].reshape(1, 1),
                p['cw1'][i].T, p['cb1'][i].reshape(1, _HID), p['cw2'][i])
            hp, xp = _sc_scatter(ma, cu, rows[k], z128, z16)
            hparts.extend([hp[0], hp[1]])
            xparts.extend([xp[0], xp[1]])
        j = min(i + 1, _L - 1)
        hcur, hrp, hcp = _tc_node(
            hcur, hparts,
            p['nw1'][i][:, :_HID].T, p['nw1'][i][:, _HID:].T,
            p['nb1'][i].reshape(1, _HID),
            p['nw2'][i].T, p['nb2'][i].reshape(1, _HID),
            p['ln_g'][i].reshape(1, _HID), p['ln_b'][i].reshape(1, _HID),
            whrT[j], whcT[j])
        for xp_part in xparts:
            xcur = xcur + xp_part

    chT16 = jnp.zeros((_HID, _XW), jnp.float32).at[:, :3].set(p['ch_w'].T)
    chb16 = jnp.zeros((1, _XW), jnp.float32).at[0, :3].set(p['ch_b'])
    hout, xd = _tc_epilogue(
        hcur,
        p['om_w1'].T, p['om_b1'].reshape(1, _HID),
        p['om_w2'].T, p['om_b2'].reshape(1, _HID),
        chT16, chb16)

    x_out = (xcur - x_init)[:_N, :3] + xd[:_N, :3]
    return (hout[:_N], x_out)


# speedup vs baseline: 2.1125x; 1.0001x over previous
import functools

import numpy as np
import jax
import jax.numpy as jnp
from jax import lax
from jax.experimental import pallas as pl
from jax.experimental.pallas import tpu as pltpu
from jax.experimental.pallas import tpu_sc as plsc

_N = 10000
_NPAD = 10240
_E = 320000
_HID = 128
_EDIM = 16
_TDIM = 64
_L = 4
_XW = 16

_NC = 2
_NS = 16
_NW = _NC * _NS

_K = 5
_EP = _E // _K
_EW = _EP // _NW
_C = 200
_NCHUNK = _EW // _C
_RPT = _NPAD // _NS

_BE = 3200
_BN = 512
_NBN = _NPAD // _BN


def _silu(v):
    return v * (1.0 / (1.0 + jnp.exp(-v)))


def _sigmoid(v):
    return 1.0 / (1.0 + jnp.exp(-v))


_TW = _HID + _XW


@functools.partial(
    pl.kernel,
    out_type=(
        jax.ShapeDtypeStruct((_EP, _HID), jnp.float32),
        jax.ShapeDtypeStruct((_EP, _HID), jnp.float32),
    ),
    mesh=plsc.VectorSubcoreMesh(core_axis_name="c", subcore_axis_name="s"),
    scratch_types=[
        pltpu.VMEM((_C,), jnp.int32),
        pltpu.VMEM((_C,), jnp.int32),
        pltpu.VMEM((_C, _TW), jnp.float32),
        pltpu.SemaphoreType.DMA,
        pltpu.SemaphoreType.DMA,
    ],
    compiler_params=pltpu.CompilerParams(use_tc_tiling_on_sc=False),
    name="egnn_sc_gather",
)
def _sc_gather(t1_h, t2_h, row_h, col_h, g_h, xd_h,
               idxr, idxc, buf, s1, s2):
    wid = lax.axis_index("s") * _NC + lax.axis_index("c")
    base0 = wid * _EW

    def body(i, carry):
        base = base0 + i * _C
        pltpu.sync_copy(row_h.at[pl.ds(base, _C)], idxr)
        pltpu.sync_copy(col_h.at[pl.ds(base, _C)], idxc)
        c1 = pltpu.async_copy(t1_h.at[idxr], buf, s1)
        c1.wait()
        c2 = pltpu.async_copy(t2_h.at[idxc], buf, s2, add=True)
        c2.wait()
        pltpu.sync_copy(buf.at[:, pl.ds(0, _HID)], g_h.at[pl.ds(base, _C)])
        pltpu.sync_copy(buf.at[:, pl.ds(_HID, _XW)],
                        xd_h.at[pl.ds(base, _C), pl.ds(0, _XW)])
        return carry

    lax.fori_loop(0, _NCHUNK, body, 0)


@functools.partial(
    pl.kernel,
    out_type=(
        jax.ShapeDtypeStruct((_NC, _NPAD, _HID), jnp.float32),
        jax.ShapeDtypeStruct((_NC, _NPAD, _XW), jnp.float32),
    ),
    mesh=plsc.VectorSubcoreMesh(core_axis_name="c", subcore_axis_name="s"),
    scratch_types=[
        pltpu.VMEM((_C,), jnp.int32),
        pltpu.VMEM((_C, _HID), jnp.float32),
        pltpu.VMEM((_C, _XW), jnp.float32),
        pltpu.VMEM_SHARED((_NPAD, _HID), jnp.float32),
        pltpu.VMEM_SHARED((_NPAD, _XW), jnp.float32),
    ],
    compiler_params=pltpu.CompilerParams(use_tc_tiling_on_sc=False),
    name="egnn_sc_scatter",
)
def _sc_scatter(ma_h, cu_h, row_h, z128_h, z16_h, hp_h, xp_h,
                idx, bufm, bufc, hacc, xacc):
    cid = lax.axis_index("c")
    sid = lax.axis_index("s")
    wid = sid * _NC + cid
    base0 = wid * _EW
    rbase = sid * _RPT

    pltpu.sync_copy(z128_h, hacc.at[pl.ds(rbase, _RPT)])
    pltpu.sync_copy(z16_h, xacc.at[pl.ds(rbase, _RPT)])
    plsc.subcore_barrier()

    def body(i, carry):
        base = base0 + i * _C
        pltpu.sync_copy(row_h.at[pl.ds(base, _C)], idx)
        pltpu.sync_copy(ma_h.at[pl.ds(base, _C)], bufm)
        pltpu.sync_copy(cu_h.at[pl.ds(base, _C), pl.ds(0, _XW)], bufc)
        pltpu.sync_copy(bufm, hacc.at[idx], add=True)
        pltpu.sync_copy(bufc, xacc.at[idx], add=True)
        return carry

    lax.fori_loop(0, _NCHUNK, body, 0)
    plsc.subcore_barrier()

    pltpu.sync_copy(hacc.at[pl.ds(rbase, _RPT)], hp_h.at[cid, pl.ds(rbase, _RPT)])
    pltpu.sync_copy(xacc.at[pl.ds(rbase, _RPT)], xp_h.at[cid, pl.ds(rbase, _RPT)])


def _tc_prologue(hpad, t11, neT, neb, tw1T, tb1, tw2T, tb2, whrT, whcT):
    def body(t_ref, h_ref, neT_ref, neb_ref, tw1T_ref, tb1_ref, tw2T_ref,
             tb2_ref, whrT_ref, whcT_ref, h0_ref, hr_ref, hc_ref):
        tval = t_ref[0, 0]
        half = _TDIM // 2
        lane_i = lax.broadcasted_iota(jnp.int32, (1, _TDIM), 1)
        lane = lane_i.astype(jnp.float32)
        k = jnp.where(lane < half, lane, lane - half)
        freq = jnp.exp(k * (-(np.log(10000.0) / (half - 1))))
        arg = tval * freq
        te0 = jnp.where(lane < half, jnp.sin(arg), jnp.cos(arg))
        te1 = _silu(jnp.dot(te0, tw1T_ref[...], preferred_element_type=jnp.float32)
                    + tb1_ref[...])
        te2 = (jnp.dot(te1, tw2T_ref[...], preferred_element_type=jnp.float32)
               + tb2_ref[...])
        h0 = (jnp.dot(h_ref[...], neT_ref[...], preferred_element_type=jnp.float32)
              + neb_ref[...] + te2)
        h0_ref[...] = h0
        hr_ref[...] = jnp.dot(h0, whrT_ref[...], preferred_element_type=jnp.float32)
        hc_ref[...] = jnp.dot(h0, whcT_ref[...], preferred_element_type=jnp.float32)

    full = lambda shape: pl.BlockSpec(shape, lambda i: (0, 0))
    return pl.pallas_call(
        body,
        grid=(_NBN,),
        in_specs=[
            pl.BlockSpec((1, 1), lambda i: (0, 0), memory_space=pltpu.SMEM),
            pl.BlockSpec((_BN, _HID), lambda i: (i, 0)),
            full((_HID, _HID)), full((1, _HID)),
            full((_TDIM, _HID)), full((1, _HID)),
            full((_HID, _HID)), full((1, _HID)),
            full((_HID, _HID)), full((_HID, _HID)),
        ],
        out_specs=[
            pl.BlockSpec((_BN, _HID), lambda i: (i, 0)),
            pl.BlockSpec((_BN, _HID), lambda i: (i, 0)),
            pl.BlockSpec((_BN, _HID), lambda i: (i, 0)),
        ],
        out_shape=[
            jax.ShapeDtypeStruct((_NPAD, _HID), jnp.float32),
            jax.ShapeDtypeStruct((_NPAD, _HID), jnp.float32),
            jax.ShapeDtypeStruct((_NPAD, _HID), jnp.float32),
        ],
        name="egnn_tc_prologue",
    )(t11, hpad, neT, neb, tw1T, tb1, tw2T, tb2, whrT, whcT)


def _tc_edge(g, xdr, ea, w_r, w_eaT, eb1, ew2T, eb2, aw, ab11,
             cw1T, cb1, cw2):
    def body(ab_ref, g_ref, xdr_ref, ea_ref, wr_ref,
             weaT_ref, eb1_ref, ew2T_ref, eb2_ref, aw_ref, cw1T_ref, cb1_ref,
             cw2_ref, ma_ref, cu_ref):
        g = g_ref[...]
        lane = lax.broadcasted_iota(jnp.int32, (_BE, _HID), 1)
        xd = jnp.where(lane < _XW, xdr_ref[...], 0.0)
        radial = jnp.sum(xd * xd, axis=-1, keepdims=True)
        pre = (g + radial * wr_ref[...]
               + jnp.dot(ea_ref[...], weaT_ref[...],
                         preferred_element_type=jnp.float32)
               + eb1_ref[...])
        m = _silu(pre)
        m = _silu(jnp.dot(m, ew2T_ref[...], preferred_element_type=jnp.float32)
                  + eb2_ref[...])
        att = _sigmoid(jnp.sum(m * aw_ref[...], axis=-1, keepdims=True)
                       + ab_ref[0, 0])
        m = m * att
        c1 = _silu(jnp.dot(m, cw1T_ref[...], preferred_element_type=jnp.float32)
                   + cb1_ref[...])
        cws = jnp.sum(c1 * cw2_ref[...], axis=-1, keepdims=True)
        cu_ref[...] = xd * (cws / jnp.sqrt(radial + 1e-08))
        ma_ref[...] = m

    full = lambda shape: pl.BlockSpec(shape, lambda i: (0, 0))
    eb = lambda w: pl.BlockSpec((_BE, w), lambda i: (i, 0))
    return pl.pallas_call(
        body,
        grid=(_EP // _BE,),
        in_specs=[
            pl.BlockSpec((1, 1), lambda i: (0, 0), memory_space=pltpu.SMEM),
            eb(_HID), eb(_HID), eb(_EDIM),
            full((1, _HID)), full((_EDIM, _HID)), full((1, _HID)),
            full((_HID, _HID)), full((1, _HID)), full((1, _HID)),
            full((_HID, _HID)), full((1, _HID)), full((1, _HID)),
        ],
        out_specs=[eb(_HID), eb(_HID)],
        out_shape=[
            jax.ShapeDtypeStruct((_EP, _HID), jnp.float32),
            jax.ShapeDtypeStruct((_EP, _HID), jnp.float32),
        ],
        name="egnn_tc_edge",
    )(ab11, g, xdr, ea, w_r, w_eaT, eb1, ew2T, eb2, aw,
      cw1T, cb1, cw2)


def _tc_node(h, hps, nw1hT, nw1mT, nb1, nw2T, nb2, ln_g, ln_b,
             whrT, whcT):
    nparts = len(hps)

    def body(*refs):
        h_ref = refs[0]
        hp_refs = refs[1:1 + nparts]
        (nw1hT_ref, nw1mT_ref, nb1_ref, nw2T_ref, nb2_ref, g_ref, b_ref,
         whrT_ref, whcT_ref, hn_ref, hr_ref, hc_ref) = refs[1 + nparts:]
        hv = h_ref[...]
        mi = hp_refs[0][...]
        for r in hp_refs[1:]:
            mi = mi + r[...]
        a = _silu(jnp.dot(hv, nw1hT_ref[...], preferred_element_type=jnp.float32)
                  + jnp.dot(mi, nw1mT_ref[...], preferred_element_type=jnp.float32)
                  + nb1_ref[...])
        hn = hv + jnp.dot(a, nw2T_ref[...], preferred_element_type=jnp.float32) \
            + nb2_ref[...]
        mu = jnp.mean(hn, axis=-1, keepdims=True)
        var = jnp.mean((hn - mu) * (hn - mu), axis=-1, keepdims=True)
        hn = (hn - mu) / jnp.sqrt(var + 1e-05) * g_ref[...] + b_ref[...]
        hn_ref[...] = hn
        hr_ref[...] = jnp.dot(hn, whrT_ref[...], preferred_element_type=jnp.float32)
        hc_ref[...] = jnp.dot(hn, whcT_ref[...], preferred_element_type=jnp.float32)

    full = lambda shape: pl.BlockSpec(shape, lambda i: (0, 0))
    nb = pl.BlockSpec((_BN, _HID), lambda i: (i, 0))
    return pl.pallas_call(
        body,
        grid=(_NBN,),
        in_specs=[nb] * (1 + nparts) + [
            full((_HID, _HID)), full((_HID, _HID)), full((1, _HID)),
            full((_HID, _HID)), full((1, _HID)), full((1, _HID)),
            full((1, _HID)), full((_HID, _HID)), full((_HID, _HID)),
        ],
        out_specs=[nb, nb, nb],
        out_shape=[
            jax.ShapeDtypeStruct((_NPAD, _HID), jnp.float32),
            jax.ShapeDtypeStruct((_NPAD, _HID), jnp.float32),
            jax.ShapeDtypeStruct((_NPAD, _HID), jnp.float32),
        ],
        name="egnn_tc_node",
    )(h, *hps, nw1hT, nw1mT, nb1, nw2T, nb2, ln_g, ln_b, whrT, whcT)


def _tc_epilogue(h, om1T, omb1, om2T, omb2, chT16, chb16):
    def body(h_ref, om1T_ref, omb1_ref, om2T_ref, omb2_ref, chT_ref, chb_ref,
             ho_ref, xd_ref):
        hv = h_ref[...]
        a = _silu(jnp.dot(hv, om1T_ref[...], preferred_element_type=jnp.float32)
                  + omb1_ref[...])
        ho_ref[...] = jnp.dot(a, om2T_ref[...], preferred_element_type=jnp.float32) \
            + omb2_ref[...]
        xd_ref[...] = jnp.dot(hv, chT_ref[...], preferred_element_type=jnp.float32) \
            + chb_ref[...]

    full = lambda shape: pl.BlockSpec(shape, lambda i: (0, 0))
    nb = pl.BlockSpec((_BN, _HID), lambda i: (i, 0))
    return pl.pallas_call(
        body,
        grid=(_NBN,),
        in_specs=[
            nb,
            full((_HID, _HID)), full((1, _HID)),
            full((_HID, _HID)), full((1, _HID)),
            full((_HID, _XW)), full((1, _XW)),
        ],
        out_specs=[nb, pl.BlockSpec((_BN, _XW), lambda i: (i, 0))],
        out_shape=[
            jax.ShapeDtypeStruct((_NPAD, _HID), jnp.float32),
            jax.ShapeDtypeStruct((_NPAD, _XW), jnp.float32),
        ],
        name="egnn_tc_epilogue",
    )(h, om1T, omb1, om2T, omb2, chT16, chb16)


def kernel(h, x, edge_index, t, edge_attr, params):
    p = params
    rows = [edge_index[0, k * _EP:(k + 1) * _EP] for k in range(_K)]
    cols = [edge_index[1, k * _EP:(k + 1) * _EP] for k in range(_K)]
    eas = [edge_attr[k * _EP:(k + 1) * _EP] for k in range(_K)]

    hpad = jnp.zeros((_NPAD, _HID), jnp.float32).at[:_N].set(h)
    xpad = jnp.zeros((_NPAD, _XW), jnp.float32).at[:_N, :3].set(x)
    x_init = xpad
    t11 = t.reshape(1, 1)
    z128 = jnp.zeros((_RPT, _HID), jnp.float32)
    z16 = jnp.zeros((_RPT, _XW), jnp.float32)

    whrT = [p['ew1'][i][:, :_HID].T for i in range(_L)]
    whcT = [p['ew1'][i][:, _HID:2 * _HID].T for i in range(_L)]
    w_r = [p['ew1'][i][:, 2 * _HID].reshape(1, _HID) for i in range(_L)]
    w_eaT = [p['ew1'][i][:, 2 * _HID + 1:].T for i in range(_L)]

    hcur, hrp, hcp = _tc_prologue(
        hpad, t11,
        p['ne_w'].T, p['ne_b'].reshape(1, _HID),
        p['te_w1'].T, p['te_b1'].reshape(1, _HID),
        p['te_w2'].T, p['te_b2'].reshape(1, _HID),
        whrT[0], whcT[0])

    xcur = xpad
    for i in range(_L):
        hparts = []
        xparts = []
        t1 = jnp.concatenate([hrp, xcur], axis=1)
        t2 = jnp.concatenate([hcp, -xcur], axis=1)
        for k in range(_K):
            g, xdr = _sc_gather(t1, t2, rows[k], cols[k])
            ma, cu = _tc_edge(
                g, xdr, eas[k],
                w_r[i], w_eaT[i], p['eb1'][i].reshape(1, _HID),
                p['ew2'][i].T, p['eb2'][i].reshape(1, _HID),
                p['aw'][i], p['ab'][i].reshape(1, 1),
                p['cw1'][i].T, p['cb1'][i].reshape(1, _HID), p['cw2'][i])
            hp, xp = _sc_scatter(ma, cu, rows[k], z128, z16)
            hparts.extend([hp[0], hp[1]])
            xparts.extend([xp[0], xp[1]])
        j = min(i + 1, _L - 1)
        hcur, hrp, hcp = _tc_node(
            hcur, hparts,
            p['nw1'][i][:, :_HID].T, p['nw1'][i][:, _HID:].T,
            p['nb1'][i].reshape(1, _HID),
            p['nw2'][i].T, p['nb2'][i].reshape(1, _HID),
            p['ln_g'][i].reshape(1, _HID), p['ln_b'][i].reshape(1, _HID),
            whrT[j], whcT[j])
        for xp_part in xparts:
            xcur = xcur + xp_part

    chT16 = jnp.zeros((_HID, _XW), jnp.float32).at[:, :3].set(p['ch_w'].T)
    chb16 = jnp.zeros((1, _XW), jnp.float32).at[0, :3].set(p['ch_b'])
    hout, xd = _tc_epilogue(
        hcur,
        p['om_w1'].T, p['om_b1'].reshape(1, _HID),
        p['om_w2'].T, p['om_b2'].reshape(1, _HID),
        chT16, chb16)

    x_out = (xcur - x_init)[:_N, :3] + xd[:_N, :3]
    return (hout[:_N], x_out)


# speedup vs baseline: 2.3195x; 1.0980x over previous
import functools

import numpy as np
import jax
import jax.numpy as jnp
from jax import lax
from jax.experimental import pallas as pl
from jax.experimental.pallas import tpu as pltpu
from jax.experimental.pallas import tpu_sc as plsc

_N = 10000
_NPAD = 10240
_E = 320000
_HID = 128
_EDIM = 16
_TDIM = 64
_L = 4
_XW = 16

_NC = 2
_NS = 16
_NW = _NC * _NS

_K = 5
_EP = _E // _K
_EW = _EP // _NW
_C = 200
_NCHUNK = _EW // _C
_RPT = _NPAD // _NS

_BE = 3200
_BN = 512
_NBN = _NPAD // _BN


def _silu(v):
    return v * (1.0 / (1.0 + jnp.exp(-v)))


def _sigmoid(v):
    return 1.0 / (1.0 + jnp.exp(-v))


_TW = _HID + _XW


@functools.partial(
    pl.kernel,
    out_type=(
        jax.ShapeDtypeStruct((_EP, _HID), jnp.float32),
        jax.ShapeDtypeStruct((_EP, _HID), jnp.float32),
    ),
    mesh=plsc.VectorSubcoreMesh(core_axis_name="c", subcore_axis_name="s"),
    scratch_types=[
        pltpu.VMEM((_C,), jnp.int32),
        pltpu.VMEM((_C,), jnp.int32),
        pltpu.VMEM((_C, _TW), jnp.float32),
        pltpu.VMEM((_C,), jnp.int32),
        pltpu.VMEM((_C,), jnp.int32),
        pltpu.VMEM((_C, _TW), jnp.float32),
        pltpu.SemaphoreType.DMA,
        pltpu.SemaphoreType.DMA,
        pltpu.SemaphoreType.DMA,
        pltpu.SemaphoreType.DMA,
        pltpu.SemaphoreType.DMA,
        pltpu.SemaphoreType.DMA,
        pltpu.SemaphoreType.DMA,
        pltpu.SemaphoreType.DMA,
        pltpu.SemaphoreType.DMA,
        pltpu.SemaphoreType.DMA,
        pltpu.SemaphoreType.DMA,
        pltpu.SemaphoreType.DMA,
    ],
    compiler_params=pltpu.CompilerParams(use_tc_tiling_on_sc=False),
    name="egnn_sc_gather",
)
def _sc_gather(t1_h, t2_h, row_h, col_h, g_h, xd_h,
               idxrA, idxcA, bufA, idxrB, idxcB, bufB,
               srA, scA, s1A, s2A, swgA, swxA,
               srB, scB, s1B, s2B, swgB, swxB):
    wid = lax.axis_index("s") * _NC + lax.axis_index("c")
    base0 = wid * _EW

    par = [(idxrA, idxcA, bufA, srA, scA, s1A, s2A, swgA, swxA),
           (idxrB, idxcB, bufB, srB, scB, s1B, s2B, swgB, swxB)]
    writes = [None, None]
    pend = None

    for i in range(_NCHUNK + 1):
        if i < _NCHUNK:
            p = i % 2
            idxr, idxc, buf, sr, sc, s1, s2, swg, swx = par[p]
            base = base0 + i * _C
            if writes[p] is not None:
                w1, w2 = writes[p]
                w1.wait()
                w2.wait()
                writes[p] = None
            lr = pltpu.async_copy(row_h.at[pl.ds(base, _C)], idxr, sr)
            lc = pltpu.async_copy(col_h.at[pl.ds(base, _C)], idxc, sc)
            lr.wait()
            c1 = pltpu.async_copy(t1_h.at[idxr], buf, s1)
        if pend is not None:
            pc2, pbuf, pbase, pswg, pswx, pp = pend
            pc2.wait()
            w1 = pltpu.async_copy(pbuf.at[:, pl.ds(0, _HID)],
                                  g_h.at[pl.ds(pbase, _C)], pswg)
            w2 = pltpu.async_copy(pbuf.at[:, pl.ds(_HID, _XW)],
                                  xd_h.at[pl.ds(pbase, _C), pl.ds(0, _XW)],
                                  pswx)
            writes[pp] = (w1, w2)
            pend = None
        if i < _NCHUNK:
            c1.wait()
            lc.wait()
            c2 = pltpu.async_copy(t2_h.at[idxc], buf, s2, add=True)
            pend = (c2, buf, base, swg, swx, p)

    for p in range(2):
        if writes[p] is not None:
            w1, w2 = writes[p]
            w1.wait()
            w2.wait()


@functools.partial(
    pl.kernel,
    out_type=(
        jax.ShapeDtypeStruct((_NC, _NPAD, _HID), jnp.float32),
        jax.ShapeDtypeStruct((_NC, _NPAD, _XW), jnp.float32),
    ),
    mesh=plsc.VectorSubcoreMesh(core_axis_name="c", subcore_axis_name="s"),
    scratch_types=[
        pltpu.VMEM((_C,), jnp.int32),
        pltpu.VMEM((_C, _HID), jnp.float32),
        pltpu.VMEM((_C, _XW), jnp.float32),
        pltpu.VMEM_SHARED((_NPAD, _HID), jnp.float32),
        pltpu.VMEM_SHARED((_NPAD, _XW), jnp.float32),
    ],
    compiler_params=pltpu.CompilerParams(use_tc_tiling_on_sc=False),
    name="egnn_sc_scatter",
)
def _sc_scatter(ma_h, cu_h, row_h, z128_h, z16_h, hp_h, xp_h,
                idx, bufm, bufc, hacc, xacc):
    cid = lax.axis_index("c")
    sid = lax.axis_index("s")
    wid = sid * _NC + cid
    base0 = wid * _EW
    rbase = sid * _RPT

    pltpu.sync_copy(z128_h, hacc.at[pl.ds(rbase, _RPT)])
    pltpu.sync_copy(z16_h, xacc.at[pl.ds(rbase, _RPT)])
    plsc.subcore_barrier()

    def body(i, carry):
        base = base0 + i * _C
        pltpu.sync_copy(row_h.at[pl.ds(base, _C)], idx)
        pltpu.sync_copy(ma_h.at[pl.ds(base, _C)], bufm)
        pltpu.sync_copy(cu_h.at[pl.ds(base, _C), pl.ds(0, _XW)], bufc)
        pltpu.sync_copy(bufm, hacc.at[idx], add=True)
        pltpu.sync_copy(bufc, xacc.at[idx], add=True)
        return carry

    lax.fori_loop(0, _NCHUNK, body, 0)
    plsc.subcore_barrier()

    pltpu.sync_copy(hacc.at[pl.ds(rbase, _RPT)], hp_h.at[cid, pl.ds(rbase, _RPT)])
    pltpu.sync_copy(xacc.at[pl.ds(rbase, _RPT)], xp_h.at[cid, pl.ds(rbase, _RPT)])


def _tc_prologue(hpad, t11, neT, neb, tw1T, tb1, tw2T, tb2, whrT, whcT):
    def body(t_ref, h_ref, neT_ref, neb_ref, tw1T_ref, tb1_ref, tw2T_ref,
             tb2_ref, whrT_ref, whcT_ref, h0_ref, hr_ref, hc_ref):
        tval = t_ref[0, 0]
        half = _TDIM // 2
        lane_i = lax.broadcasted_iota(jnp.int32, (1, _TDIM), 1)
        lane = lane_i.astype(jnp.float32)
        k = jnp.where(lane < half, lane, lane - half)
        freq = jnp.exp(k * (-(np.log(10000.0) / (half - 1))))
        arg = tval * freq
        te0 = jnp.where(lane < half, jnp.sin(arg), jnp.cos(arg))
        te1 = _silu(jnp.dot(te0, tw1T_ref[...], preferred_element_type=jnp.float32)
                    + tb1_ref[...])
        te2 = (jnp.dot(te1, tw2T_ref[...], preferred_element_type=jnp.float32)
               + tb2_ref[...])
        h0 = (jnp.dot(h_ref[...], neT_ref[...], preferred_element_type=jnp.float32)
              + neb_ref[...] + te2)
        h0_ref[...] = h0
        hr_ref[...] = jnp.dot(h0, whrT_ref[...], preferred_element_type=jnp.float32)
        hc_ref[...] = jnp.dot(h0, whcT_ref[...], preferred_element_type=jnp.float32)

    full = lambda shape: pl.BlockSpec(shape, lambda i: (0, 0))
    return pl.pallas_call(
        body,
        grid=(_NBN,),
        in_specs=[
            pl.BlockSpec((1, 1), lambda i: (0, 0), memory_space=pltpu.SMEM),
            pl.BlockSpec((_BN, _HID), lambda i: (i, 0)),
            full((_HID, _HID)), full((1, _HID)),
            full((_TDIM, _HID)), full((1, _HID)),
            full((_HID, _HID)), full((1, _HID)),
            full((_HID, _HID)), full((_HID, _HID)),
        ],
        out_specs=[
            pl.BlockSpec((_BN, _HID), lambda i: (i, 0)),
            pl.BlockSpec((_BN, _HID), lambda i: (i, 0)),
            pl.BlockSpec((_BN, _HID), lambda i: (i, 0)),
        ],
        out_shape=[
            jax.ShapeDtypeStruct((_NPAD, _HID), jnp.float32),
            jax.ShapeDtypeStruct((_NPAD, _HID), jnp.float32),
            jax.ShapeDtypeStruct((_NPAD, _HID), jnp.float32),
        ],
        name="egnn_tc_prologue",
    )(t11, hpad, neT, neb, tw1T, tb1, tw2T, tb2, whrT, whcT)


def _tc_edge(g, xdr, ea, w_r, w_eaT, eb1, ew2T, eb2, aw, ab11,
             cw1T, cb1, cw2):
    def body(ab_ref, g_ref, xdr_ref, ea_ref, wr_ref,
             weaT_ref, eb1_ref, ew2T_ref, eb2_ref, aw_ref, cw1T_ref, cb1_ref,
             cw2_ref, ma_ref, cu_ref):
        g = g_ref[...]
        lane = lax.broadcasted_iota(jnp.int32, (_BE, _HID), 1)
        xd = jnp.where(lane < _XW, xdr_ref[...], 0.0)
        radial = jnp.sum(xd * xd, axis=-1, keepdims=True)
        pre = (g + radial * wr_ref[...]
               + jnp.dot(ea_ref[...], weaT_ref[...],
                         preferred_element_type=jnp.float32)
               + eb1_ref[...])
        m = _silu(pre)
        m = _silu(jnp.dot(m, ew2T_ref[...], preferred_element_type=jnp.float32)
                  + eb2_ref[...])
        att = _sigmoid(jnp.sum(m * aw_ref[...], axis=-1, keepdims=True)
                       + ab_ref[0, 0])
        m = m * att
        c1 = _silu(jnp.dot(m, cw1T_ref[...], preferred_element_type=jnp.float32)
                   + cb1_ref[...])
        cws = jnp.sum(c1 * cw2_ref[...], axis=-1, keepdims=True)
        cu_ref[...] = xd * (cws / jnp.sqrt(radial + 1e-08))
        ma_ref[...] = m

    full = lambda shape: pl.BlockSpec(shape, lambda i: (0, 0))
    eb = lambda w: pl.BlockSpec((_BE, w), lambda i: (i, 0))
    return pl.pallas_call(
        body,
        grid=(_EP // _BE,),
        in_specs=[
            pl.BlockSpec((1, 1), lambda i: (0, 0), memory_space=pltpu.SMEM),
            eb(_HID), eb(_HID), eb(_EDIM),
            full((1, _HID)), full((_EDIM, _HID)), full((1, _HID)),
            full((_HID, _HID)), full((1, _HID)), full((1, _HID)),
            full((_HID, _HID)), full((1, _HID)), full((1, _HID)),
        ],
        out_specs=[eb(_HID), eb(_HID)],
        out_shape=[
            jax.ShapeDtypeStruct((_EP, _HID), jnp.float32),
            jax.ShapeDtypeStruct((_EP, _HID), jnp.float32),
        ],
        name="egnn_tc_edge",
    )(ab11, g, xdr, ea, w_r, w_eaT, eb1, ew2T, eb2, aw,
      cw1T, cb1, cw2)


def _tc_node(h, hps, nw1hT, nw1mT, nb1, nw2T, nb2, ln_g, ln_b,
             whrT, whcT):
    nparts = len(hps)

    def body(*refs):
        h_ref = refs[0]
        hp_refs = refs[1:1 + nparts]
        (nw1hT_ref, nw1mT_ref, nb1_ref, nw2T_ref, nb2_ref, g_ref, b_ref,
         whrT_ref, whcT_ref, hn_ref, hr_ref, hc_ref) = refs[1 + nparts:]
        hv = h_ref[...]
        mi = hp_refs[0][...]
        for r in hp_refs[1:]:
            mi = mi + r[...]
        a = _silu(jnp.dot(hv, nw1hT_ref[...], preferred_element_type=jnp.float32)
                  + jnp.dot(mi, nw1mT_ref[...], preferred_element_type=jnp.float32)
                  + nb1_ref[...])
        hn = hv + jnp.dot(a, nw2T_ref[...], preferred_element_type=jnp.float32) \
            + nb2_ref[...]
        mu = jnp.mean(hn, axis=-1, keepdims=True)
        var = jnp.mean((hn - mu) * (hn - mu), axis=-1, keepdims=True)
        hn = (hn - mu) / jnp.sqrt(var + 1e-05) * g_ref[...] + b_ref[...]
        hn_ref[...] = hn
        hr_ref[...] = jnp.dot(hn, whrT_ref[...], preferred_element_type=jnp.float32)
        hc_ref[...] = jnp.dot(hn, whcT_ref[...], preferred_element_type=jnp.float32)

    full = lambda shape: pl.BlockSpec(shape, lambda i: (0, 0))
    nb = pl.BlockSpec((_BN, _HID), lambda i: (i, 0))
    return pl.pallas_call(
        body,
        grid=(_NBN,),
        in_specs=[nb] * (1 + nparts) + [
            full((_HID, _HID)), full((_HID, _HID)), full((1, _HID)),
            full((_HID, _HID)), full((1, _HID)), full((1, _HID)),
            full((1, _HID)), full((_HID, _HID)), full((_HID, _HID)),
        ],
        out_specs=[nb, nb, nb],
        out_shape=[
            jax.ShapeDtypeStruct((_NPAD, _HID), jnp.float32),
            jax.ShapeDtypeStruct((_NPAD, _HID), jnp.float32),
            jax.ShapeDtypeStruct((_NPAD, _HID), jnp.float32),
        ],
        name="egnn_tc_node",
    )(h, *hps, nw1hT, nw1mT, nb1, nw2T, nb2, ln_g, ln_b, whrT, whcT)


def _tc_epilogue(h, om1T, omb1, om2T, omb2, chT16, chb16):
    def body(h_ref, om1T_ref, omb1_ref, om2T_ref, omb2_ref, chT_ref, chb_ref,
             ho_ref, xd_ref):
        hv = h_ref[...]
        a = _silu(jnp.dot(hv, om1T_ref[...], preferred_element_type=jnp.float32)
                  + omb1_ref[...])
        ho_ref[...] = jnp.dot(a, om2T_ref[...], preferred_element_type=jnp.float32) \
            + omb2_ref[...]
        xd_ref[...] = jnp.dot(hv, chT_ref[...], preferred_element_type=jnp.float32) \
            + chb_ref[...]

    full = lambda shape: pl.BlockSpec(shape, lambda i: (0, 0))
    nb = pl.BlockSpec((_BN, _HID), lambda i: (i, 0))
    return pl.pallas_call(
        body,
        grid=(_NBN,),
        in_specs=[
            nb,
            full((_HID, _HID)), full((1, _HID)),
            full((_HID, _HID)), full((1, _HID)),
            full((_HID, _XW)), full((1, _XW)),
        ],
        out_specs=[nb, pl.BlockSpec((_BN, _XW), lambda i: (i, 0))],
        out_shape=[
            jax.ShapeDtypeStruct((_NPAD, _HID), jnp.float32),
            jax.ShapeDtypeStruct((_NPAD, _XW), jnp.float32),
        ],
        name="egnn_tc_epilogue",
    )(h, om1T, omb1, om2T, omb2, chT16, chb16)


def kernel(h, x, edge_index, t, edge_attr, params):
    p = params
    rows = [edge_index[0, k * _EP:(k + 1) * _EP] for k in range(_K)]
    cols = [edge_index[1, k * _EP:(k + 1) * _EP] for k in range(_K)]
    eas = [edge_attr[k * _EP:(k + 1) * _EP] for k in range(_K)]

    hpad = jnp.zeros((_NPAD, _HID), jnp.float32).at[:_N].set(h)
    xpad = jnp.zeros((_NPAD, _XW), jnp.float32).at[:_N, :3].set(x)
    x_init = xpad
    t11 = t.reshape(1, 1)
    z128 = jnp.zeros((_RPT, _HID), jnp.float32)
    z16 = jnp.zeros((_RPT, _XW), jnp.float32)

    whrT = [p['ew1'][i][:, :_HID].T for i in range(_L)]
    whcT = [p['ew1'][i][:, _HID:2 * _HID].T for i in range(_L)]
    w_r = [p['ew1'][i][:, 2 * _HID].reshape(1, _HID) for i in range(_L)]
    w_eaT = [p['ew1'][i][:, 2 * _HID + 1:].T for i in range(_L)]

    hcur, hrp, hcp = _tc_prologue(
        hpad, t11,
        p['ne_w'].T, p['ne_b'].reshape(1, _HID),
        p['te_w1'].T, p['te_b1'].reshape(1, _HID),
        p['te_w2'].T, p['te_b2'].reshape(1, _HID),
        whrT[0], whcT[0])

    xcur = xpad
    for i in range(_L):
        hparts = []
        xparts = []
        t1 = jnp.concatenate([hrp, xcur], axis=1)
        t2 = jnp.concatenate([hcp, -xcur], axis=1)
        for k in range(_K):
            g, xdr = _sc_gather(t1, t2, rows[k], cols[k])
            ma, cu = _tc_edge(
                g, xdr, eas[k],
                w_r[i], w_eaT[i], p['eb1'][i].reshape(1, _HID),
                p['ew2'][i].T, p['eb2'][i].reshape(1, _HID),
                p['aw'][i], p['ab'][i].reshape(1, 1),
                p['cw1'][i].T, p['cb1'][i].reshape(1, _HID), p['cw2'][i])
            hp, xp = _sc_scatter(ma, cu, rows[k], z128, z16)
            hparts.extend([hp[0], hp[1]])
            xparts.extend([xp[0], xp[1]])
        j = min(i + 1, _L - 1)
        hcur, hrp, hcp = _tc_node(
            hcur, hparts,
            p['nw1'][i][:, :_HID].T, p['nw1'][i][:, _HID:].T,
            p['nb1'][i].reshape(1, _HID),
            p['nw2'][i].T, p['nb2'][i].reshape(1, _HID),
            p['ln_g'][i].reshape(1, _HID), p['ln_b'][i].reshape(1, _HID),
            whrT[j], whcT[j])
        for xp_part in xparts:
            xcur = xcur + xp_part

    chT16 = jnp.zeros((_HID, _XW), jnp.float32).at[:, :3].set(p['ch_w'].T)
    chb16 = jnp.zeros((1, _XW), jnp.float32).at[0, :3].set(p['ch_b'])
    hout, xd = _tc_epilogue(
        hcur,
        p['om_w1'].T, p['om_b1'].reshape(1, _HID),
        p['om_w2'].T, p['om_b2'].reshape(1, _HID),
        chT16, chb16)

    x_out = (xcur - x_init)[:_N, :3] + xd[:_N, :3]
    return (hout[:_N], x_out)


# speedup vs baseline: 2.5171x; 1.0852x over previous
import functools

import numpy as np
import jax
import jax.numpy as jnp
from jax import lax
from jax.experimental import pallas as pl
from jax.experimental.pallas import tpu as pltpu
from jax.experimental.pallas import tpu_sc as plsc

_N = 10000
_NPAD = 10240
_E = 320000
_HID = 128
_EDIM = 16
_TDIM = 64
_L = 4
_XW = 16

_NC = 2
_NS = 16
_NW = _NC * _NS

_K = 5
_EP = _E // _K
_EW = _EP // _NW
_C = 200
_NCHUNK = _EW // _C
_CS = 80
_NCHUNKS = _EW // _CS
_RPT = _NPAD // _NS

_BE = 3200
_BN = 512
_NBN = _NPAD // _BN


def _silu(v):
    return v * (1.0 / (1.0 + jnp.exp(-v)))


def _sigmoid(v):
    return 1.0 / (1.0 + jnp.exp(-v))


_TW = _HID + _XW


@functools.partial(
    pl.kernel,
    out_type=(
        jax.ShapeDtypeStruct((_EP, _HID), jnp.float32),
        jax.ShapeDtypeStruct((_EP, _HID), jnp.float32),
    ),
    mesh=plsc.VectorSubcoreMesh(core_axis_name="c", subcore_axis_name="s"),
    scratch_types=[
        pltpu.VMEM((_C,), jnp.int32),
        pltpu.VMEM((_C,), jnp.int32),
        pltpu.VMEM((_C, _TW), jnp.float32),
        pltpu.VMEM((_C,), jnp.int32),
        pltpu.VMEM((_C,), jnp.int32),
        pltpu.VMEM((_C, _TW), jnp.float32),
        pltpu.SemaphoreType.DMA,
        pltpu.SemaphoreType.DMA,
        pltpu.SemaphoreType.DMA,
        pltpu.SemaphoreType.DMA,
        pltpu.SemaphoreType.DMA,
        pltpu.SemaphoreType.DMA,
        pltpu.SemaphoreType.DMA,
        pltpu.SemaphoreType.DMA,
        pltpu.SemaphoreType.DMA,
        pltpu.SemaphoreType.DMA,
        pltpu.SemaphoreType.DMA,
        pltpu.SemaphoreType.DMA,
    ],
    compiler_params=pltpu.CompilerParams(use_tc_tiling_on_sc=False),
    name="egnn_sc_gather",
)
def _sc_gather(t1_h, t2_h, row_h, col_h, g_h, xd_h,
               idxrA, idxcA, bufA, idxrB, idxcB, bufB,
               srA, scA, s1A, s2A, swgA, swxA,
               srB, scB, s1B, s2B, swgB, swxB):
    wid = lax.axis_index("s") * _NC + lax.axis_index("c")
    base0 = wid * _EW

    par = [(idxrA, idxcA, bufA, srA, scA, s1A, s2A, swgA, swxA),
           (idxrB, idxcB, bufB, srB, scB, s1B, s2B, swgB, swxB)]
    writes = [None, None]
    pend = None

    for i in range(_NCHUNK + 1):
        if i < _NCHUNK:
            p = i % 2
            idxr, idxc, buf, sr, sc, s1, s2, swg, swx = par[p]
            base = base0 + i * _C
            if writes[p] is not None:
                w1, w2 = writes[p]
                w1.wait()
                w2.wait()
                writes[p] = None
            lr = pltpu.async_copy(row_h.at[pl.ds(base, _C)], idxr, sr)
            lc = pltpu.async_copy(col_h.at[pl.ds(base, _C)], idxc, sc)
            lr.wait()
            c1 = pltpu.async_copy(t1_h.at[idxr], buf, s1)
        if pend is not None:
            pc2, pbuf, pbase, pswg, pswx, pp = pend
            pc2.wait()
            w1 = pltpu.async_copy(pbuf.at[:, pl.ds(0, _HID)],
                                  g_h.at[pl.ds(pbase, _C)], pswg)
            w2 = pltpu.async_copy(pbuf.at[:, pl.ds(_HID, _XW)],
                                  xd_h.at[pl.ds(pbase, _C), pl.ds(0, _XW)],
                                  pswx)
            writes[pp] = (w1, w2)
            pend = None
        if i < _NCHUNK:
            c1.wait()
            lc.wait()
            c2 = pltpu.async_copy(t2_h.at[idxc], buf, s2, add=True)
            pend = (c2, buf, base, swg, swx, p)

    for p in range(2):
        if writes[p] is not None:
            w1, w2 = writes[p]
            w1.wait()
            w2.wait()


@functools.partial(
    pl.kernel,
    out_type=(
        jax.ShapeDtypeStruct((_NC, _NPAD, _HID), jnp.float32),
        jax.ShapeDtypeStruct((_NC, _NPAD, _XW), jnp.float32),
    ),
    mesh=plsc.VectorSubcoreMesh(core_axis_name="c", subcore_axis_name="s"),
    scratch_types=[
        pltpu.VMEM((_CS,), jnp.int32),
        pltpu.VMEM((_CS, _HID), jnp.float32),
        pltpu.VMEM((_CS, _XW), jnp.float32),
        pltpu.VMEM((_CS,), jnp.int32),
        pltpu.VMEM((_CS, _HID), jnp.float32),
        pltpu.VMEM((_CS, _XW), jnp.float32),
        pltpu.VMEM_SHARED((_NPAD, _HID), jnp.float32),
        pltpu.VMEM_SHARED((_NPAD, _XW), jnp.float32),
        pltpu.SemaphoreType.DMA,
        pltpu.SemaphoreType.DMA,
        pltpu.SemaphoreType.DMA,
        pltpu.SemaphoreType.DMA,
        pltpu.SemaphoreType.DMA,
        pltpu.SemaphoreType.DMA,
        pltpu.SemaphoreType.DMA,
        pltpu.SemaphoreType.DMA,
        pltpu.SemaphoreType.DMA,
        pltpu.SemaphoreType.DMA,
    ],
    compiler_params=pltpu.CompilerParams(use_tc_tiling_on_sc=False),
    name="egnn_sc_scatter",
)
def _sc_scatter(ma_h, cu_h, row_h, z128_h, z16_h, hp_h, xp_h,
                idxA, bufmA, bufcA, idxB, bufmB, bufcB, hacc, xacc,
                liA, lmA, lcA, ahA, axA, liB, lmB, lcB, ahB, axB):
    cid = lax.axis_index("c")
    sid = lax.axis_index("s")
    wid = sid * _NC + cid
    base0 = wid * _EW
    rbase = sid * _RPT

    pltpu.sync_copy(z128_h, hacc.at[pl.ds(rbase, _RPT)])
    pltpu.sync_copy(z16_h, xacc.at[pl.ds(rbase, _RPT)])
    plsc.subcore_barrier()

    par = [(idxA, bufmA, bufcA, liA, lmA, lcA, ahA, axA),
           (idxB, bufmB, bufcB, liB, lmB, lcB, ahB, axB)]
    adds = [None, None]
    pend = None

    for i in range(_NCHUNKS + 1):
        if i < _NCHUNKS:
            p = i % 2
            idx, bufm, bufc, li, lm, lc, ah, ax = par[p]
            base = base0 + i * _CS
            if adds[p] is not None:
                a1, a2 = adds[p]
                a1.wait()
                a2.wait()
                adds[p] = None
            l1 = pltpu.async_copy(row_h.at[pl.ds(base, _CS)], idx, li)
            l2 = pltpu.async_copy(ma_h.at[pl.ds(base, _CS)], bufm, lm)
            l3 = pltpu.async_copy(cu_h.at[pl.ds(base, _CS), pl.ds(0, _XW)],
                                  bufc, lc)
        if pend is not None:
            pl1, pl2, pl3, pp = pend
            pl1.wait()
            pl2.wait()
            pl3.wait()
            pidx, pbufm, pbufc, _, _, _, pah, pax = par[pp]
            a1 = pltpu.async_copy(pbufm, hacc.at[pidx], pah, add=True)
            a2 = pltpu.async_copy(pbufc, xacc.at[pidx], pax, add=True)
            adds[pp] = (a1, a2)
            pend = None
        if i < _NCHUNKS:
            pend = (l1, l2, l3, p)

    for p in range(2):
        if adds[p] is not None:
            a1, a2 = adds[p]
            a1.wait()
            a2.wait()
    plsc.subcore_barrier()

    pltpu.sync_copy(hacc.at[pl.ds(rbase, _RPT)], hp_h.at[cid, pl.ds(rbase, _RPT)])
    pltpu.sync_copy(xacc.at[pl.ds(rbase, _RPT)], xp_h.at[cid, pl.ds(rbase, _RPT)])


def _tc_prologue(hpad, t11, neT, neb, tw1T, tb1, tw2T, tb2, whrT, whcT):
    def body(t_ref, h_ref, neT_ref, neb_ref, tw1T_ref, tb1_ref, tw2T_ref,
             tb2_ref, whrT_ref, whcT_ref, h0_ref, hr_ref, hc_ref):
        tval = t_ref[0, 0]
        half = _TDIM // 2
        lane_i = lax.broadcasted_iota(jnp.int32, (1, _TDIM), 1)
        lane = lane_i.astype(jnp.float32)
        k = jnp.where(lane < half, lane, lane - half)
        freq = jnp.exp(k * (-(np.log(10000.0) / (half - 1))))
        arg = tval * freq
        te0 = jnp.where(lane < half, jnp.sin(arg), jnp.cos(arg))
        te1 = _silu(jnp.dot(te0, tw1T_ref[...], preferred_element_type=jnp.float32)
                    + tb1_ref[...])
        te2 = (jnp.dot(te1, tw2T_ref[...], preferred_element_type=jnp.float32)
               + tb2_ref[...])
        h0 = (jnp.dot(h_ref[...], neT_ref[...], preferred_element_type=jnp.float32)
              + neb_ref[...] + te2)
        h0_ref[...] = h0
        hr_ref[...] = jnp.dot(h0, whrT_ref[...], preferred_element_type=jnp.float32)
        hc_ref[...] = jnp.dot(h0, whcT_ref[...], preferred_element_type=jnp.float32)

    full = lambda shape: pl.BlockSpec(shape, lambda i: (0, 0))
    return pl.pallas_call(
        body,
        grid=(_NBN,),
        in_specs=[
            pl.BlockSpec((1, 1), lambda i: (0, 0), memory_space=pltpu.SMEM),
            pl.BlockSpec((_BN, _HID), lambda i: (i, 0)),
            full((_HID, _HID)), full((1, _HID)),
            full((_TDIM, _HID)), full((1, _HID)),
            full((_HID, _HID)), full((1, _HID)),
            full((_HID, _HID)), full((_HID, _HID)),
        ],
        out_specs=[
            pl.BlockSpec((_BN, _HID), lambda i: (i, 0)),
            pl.BlockSpec((_BN, _HID), lambda i: (i, 0)),
            pl.BlockSpec((_BN, _HID), lambda i: (i, 0)),
        ],
        out_shape=[
            jax.ShapeDtypeStruct((_NPAD, _HID), jnp.float32),
            jax.ShapeDtypeStruct((_NPAD, _HID), jnp.float32),
            jax.ShapeDtypeStruct((_NPAD, _HID), jnp.float32),
        ],
        name="egnn_tc_prologue",
    )(t11, hpad, neT, neb, tw1T, tb1, tw2T, tb2, whrT, whcT)


def _tc_edge(g, xdr, ea, w_r, w_eaT, eb1, ew2T, eb2, aw, ab11,
             cw1T, cb1, cw2):
    def body(ab_ref, g_ref, xdr_ref, ea_ref, wr_ref,
             weaT_ref, eb1_ref, ew2T_ref, eb2_ref, aw_ref, cw1T_ref, cb1_ref,
             cw2_ref, ma_ref, cu_ref):
        g = g_ref[...]
        lane = lax.broadcasted_iota(jnp.int32, (_BE, _HID), 1)
        xd = jnp.where(lane < _XW, xdr_ref[...], 0.0)
        radial = jnp.sum(xd * xd, axis=-1, keepdims=True)
        pre = (g + radial * wr_ref[...]
               + jnp.dot(ea_ref[...], weaT_ref[...],
                         preferred_element_type=jnp.float32)
               + eb1_ref[...])
        m = _silu(pre)
        m = _silu(jnp.dot(m, ew2T_ref[...], preferred_element_type=jnp.float32)
                  + eb2_ref[...])
        att = _sigmoid(jnp.sum(m * aw_ref[...], axis=-1, keepdims=True)
                       + ab_ref[0, 0])
        m = m * att
        c1 = _silu(jnp.dot(m, cw1T_ref[...], preferred_element_type=jnp.float32)
                   + cb1_ref[...])
        cws = jnp.sum(c1 * cw2_ref[...], axis=-1, keepdims=True)
        cu_ref[...] = xd * (cws / jnp.sqrt(radial + 1e-08))
        ma_ref[...] = m

    full = lambda shape: pl.BlockSpec(shape, lambda i: (0, 0))
    eb = lambda w: pl.BlockSpec((_BE, w), lambda i: (i, 0))
    return pl.pallas_call(
        body,
        grid=(_EP // _BE,),
        in_specs=[
            pl.BlockSpec((1, 1), lambda i: (0, 0), memory_space=pltpu.SMEM),
            eb(_HID), eb(_HID), eb(_EDIM),
            full((1, _HID)), full((_EDIM, _HID)), full((1, _HID)),
            full((_HID, _HID)), full((1, _HID)), full((1, _HID)),
            full((_HID, _HID)), full((1, _HID)), full((1, _HID)),
        ],
        out_specs=[eb(_HID), eb(_HID)],
        out_shape=[
            jax.ShapeDtypeStruct((_EP, _HID), jnp.float32),
            jax.ShapeDtypeStruct((_EP, _HID), jnp.float32),
        ],
        name="egnn_tc_edge",
    )(ab11, g, xdr, ea, w_r, w_eaT, eb1, ew2T, eb2, aw,
      cw1T, cb1, cw2)


def _tc_node(h, hps, nw1hT, nw1mT, nb1, nw2T, nb2, ln_g, ln_b,
             whrT, whcT):
    nparts = len(hps)

    def body(*refs):
        h_ref = refs[0]
        hp_refs = refs[1:1 + nparts]
        (nw1hT_ref, nw1mT_ref, nb1_ref, nw2T_ref, nb2_ref, g_ref, b_ref,
         whrT_ref, whcT_ref, hn_ref, hr_ref, hc_ref) = refs[1 + nparts:]
        hv = h_ref[...]
        mi = hp_refs[0][...]
        for r in hp_refs[1:]:
            mi = mi + r[...]
        a = _silu(jnp.dot(hv, nw1hT_ref[...], preferred_element_type=jnp.float32)
                  + jnp.dot(mi, nw1mT_ref[...], preferred_element_type=jnp.float32)
                  + nb1_ref[...])
        hn = hv + jnp.dot(a, nw2T_ref[...], preferred_element_type=jnp.float32) \
            + nb2_ref[...]
        mu = jnp.mean(hn, axis=-1, keepdims=True)
        var = jnp.mean((hn - mu) * (hn - mu), axis=-1, keepdims=True)
        hn = (hn - mu) / jnp.sqrt(var + 1e-05) * g_ref[...] + b_ref[...]
        hn_ref[...] = hn
        hr_ref[...] = jnp.dot(hn, whrT_ref[...], preferred_element_type=jnp.float32)
        hc_ref[...] = jnp.dot(hn, whcT_ref[...], preferred_element_type=jnp.float32)

    full = lambda shape: pl.BlockSpec(shape, lambda i: (0, 0))
    nb = pl.BlockSpec((_BN, _HID), lambda i: (i, 0))
    return pl.pallas_call(
        body,
        grid=(_NBN,),
        in_specs=[nb] * (1 + nparts) + [
            full((_HID, _HID)), full((_HID, _HID)), full((1, _HID)),
            full((_HID, _HID)), full((1, _HID)), full((1, _HID)),
            full((1, _HID)), full((_HID, _HID)), full((_HID, _HID)),
        ],
        out_specs=[nb, nb, nb],
        out_shape=[
            jax.ShapeDtypeStruct((_NPAD, _HID), jnp.float32),
            jax.ShapeDtypeStruct((_NPAD, _HID), jnp.float32),
            jax.ShapeDtypeStruct((_NPAD, _HID), jnp.float32),
        ],
        name="egnn_tc_node",
    )(h, *hps, nw1hT, nw1mT, nb1, nw2T, nb2, ln_g, ln_b, whrT, whcT)


def _tc_epilogue(h, om1T, omb1, om2T, omb2, chT16, chb16):
    def body(h_ref, om1T_ref, omb1_ref, om2T_ref, omb2_ref, chT_ref, chb_ref,
             ho_ref, xd_ref):
        hv = h_ref[...]
        a = _silu(jnp.dot(hv, om1T_ref[...], preferred_element_type=jnp.float32)
                  + omb1_ref[...])
        ho_ref[...] = jnp.dot(a, om2T_ref[...], preferred_element_type=jnp.float32) \
            + omb2_ref[...]
        xd_ref[...] = jnp.dot(hv, chT_ref[...], preferred_element_type=jnp.float32) \
            + chb_ref[...]

    full = lambda shape: pl.BlockSpec(shape, lambda i: (0, 0))
    nb = pl.BlockSpec((_BN, _HID), lambda i: (i, 0))
    return pl.pallas_call(
        body,
        grid=(_NBN,),
        in_specs=[
            nb,
            full((_HID, _HID)), full((1, _HID)),
            full((_HID, _HID)), full((1, _HID)),
            full((_HID, _XW)), full((1, _XW)),
        ],
        out_specs=[nb, pl.BlockSpec((_BN, _XW), lambda i: (i, 0))],
        out_shape=[
            jax.ShapeDtypeStruct((_NPAD, _HID), jnp.float32),
            jax.ShapeDtypeStruct((_NPAD, _XW), jnp.float32),
        ],
        name="egnn_tc_epilogue",
    )(h, om1T, omb1, om2T, omb2, chT16, chb16)


def kernel(h, x, edge_index, t, edge_attr, params):
    p = params
    rows = [edge_index[0, k * _EP:(k + 1) * _EP] for k in range(_K)]
    cols = [edge_index[1, k * _EP:(k + 1) * _EP] for k in range(_K)]
    eas = [edge_attr[k * _EP:(k + 1) * _EP] for k in range(_K)]

    hpad = jnp.zeros((_NPAD, _HID), jnp.float32).at[:_N].set(h)
    xpad = jnp.zeros((_NPAD, _XW), jnp.float32).at[:_N, :3].set(x)
    x_init = xpad
    t11 = t.reshape(1, 1)
    z128 = jnp.zeros((_RPT, _HID), jnp.float32)
    z16 = jnp.zeros((_RPT, _XW), jnp.float32)

    whrT = [p['ew1'][i][:, :_HID].T for i in range(_L)]
    whcT = [p['ew1'][i][:, _HID:2 * _HID].T for i in range(_L)]
    w_r = [p['ew1'][i][:, 2 * _HID].reshape(1, _HID) for i in range(_L)]
    w_eaT = [p['ew1'][i][:, 2 * _HID + 1:].T for i in range(_L)]

    hcur, hrp, hcp = _tc_prologue(
        hpad, t11,
        p['ne_w'].T, p['ne_b'].reshape(1, _HID),
        p['te_w1'].T, p['te_b1'].reshape(1, _HID),
        p['te_w2'].T, p['te_b2'].reshape(1, _HID),
        whrT[0], whcT[0])

    xcur = xpad
    for i in range(_L):
        hparts = []
        xparts = []
        t1 = jnp.concatenate([hrp, xcur], axis=1)
        t2 = jnp.concatenate([hcp, -xcur], axis=1)
        for k in range(_K):
            g, xdr = _sc_gather(t1, t2, rows[k], cols[k])
            ma, cu = _tc_edge(
                g, xdr, eas[k],
                w_r[i], w_eaT[i], p['eb1'][i].reshape(1, _HID),
                p['ew2'][i].T, p['eb2'][i].reshape(1, _HID),
                p['aw'][i], p['ab'][i].reshape(1, 1),
                p['cw1'][i].T, p['cb1'][i].reshape(1, _HID), p['cw2'][i])
            hp, xp = _sc_scatter(ma, cu, rows[k], z128, z16)
            hparts.extend([hp[0], hp[1]])
            xparts.extend([xp[0], xp[1]])
        j = min(i + 1, _L - 1)
        hcur, hrp, hcp = _tc_node(
            hcur, hparts,
            p['nw1'][i][:, :_HID].T, p['nw1'][i][:, _HID:].T,
            p['nb1'][i].reshape(1, _HID),
            p['nw2'][i].T, p['nb2'][i].reshape(1, _HID),
            p['ln_g'][i].reshape(1, _HID), p['ln_b'][i].reshape(1, _HID),
            whrT[j], whcT[j])
        for xp_part in xparts:
            xcur = xcur + xp_part

    chT16 = jnp.zeros((_HID, _XW), jnp.float32).at[:, :3].set(p['ch_w'].T)
    chb16 = jnp.zeros((1, _XW), jnp.float32).at[0, :3].set(p['ch_b'])
    hout, xd = _tc_epilogue(
        hcur,
        p['om_w1'].T, p['om_b1'].reshape(1, _HID),
        p['om_w2'].T, p['om_b2'].reshape(1, _HID),
        chT16, chb16)

    x_out = (xcur - x_init)[:_N, :3] + xd[:_N, :3]
    return (hout[:_N], x_out)


# speedup vs baseline: 2.7402x; 1.0886x over previous
import functools

import numpy as np
import jax
import jax.numpy as jnp
from jax import lax
from jax.experimental import pallas as pl
from jax.experimental.pallas import tpu as pltpu
from jax.experimental.pallas import tpu_sc as plsc

_N = 10000
_NPAD = 10240
_E = 320000
_HID = 128
_EDIM = 16
_TDIM = 64
_L = 4
_XW = 16

_NC = 2
_NS = 16
_NW = _NC * _NS

_K = 2
_EP = _E // _K
_EW = _EP // _NW
_C = 200
_NCHUNK = _EW // _C
_CS = 40
_NCHUNKS = _EW // _CS
_RPT = _NPAD // _NS

_BE = 3200
_BN = 512
_NBN = _NPAD // _BN


def _silu(v):
    return v * (1.0 / (1.0 + jnp.exp(-v)))


def _sigmoid(v):
    return 1.0 / (1.0 + jnp.exp(-v))


_TW = _HID + _XW


@functools.partial(
    pl.kernel,
    out_type=(
        jax.ShapeDtypeStruct((_EP, _HID), jnp.float32),
        jax.ShapeDtypeStruct((_EP, _HID), jnp.float32),
    ),
    mesh=plsc.VectorSubcoreMesh(core_axis_name="c", subcore_axis_name="s"),
    scratch_types=[
        pltpu.VMEM((_C,), jnp.int32),
        pltpu.VMEM((_C,), jnp.int32),
        pltpu.VMEM((_C, _TW), jnp.float32),
        pltpu.VMEM((_C,), jnp.int32),
        pltpu.VMEM((_C,), jnp.int32),
        pltpu.VMEM((_C, _TW), jnp.float32),
        pltpu.SemaphoreType.DMA,
        pltpu.SemaphoreType.DMA,
        pltpu.SemaphoreType.DMA,
        pltpu.SemaphoreType.DMA,
        pltpu.SemaphoreType.DMA,
        pltpu.SemaphoreType.DMA,
        pltpu.SemaphoreType.DMA,
        pltpu.SemaphoreType.DMA,
        pltpu.SemaphoreType.DMA,
        pltpu.SemaphoreType.DMA,
        pltpu.SemaphoreType.DMA,
        pltpu.SemaphoreType.DMA,
    ],
    compiler_params=pltpu.CompilerParams(use_tc_tiling_on_sc=False),
    name="egnn_sc_gather",
)
def _sc_gather(t1_h, t2_h, row_h, col_h, g_h, xd_h,
               idxrA, idxcA, bufA, idxrB, idxcB, bufB,
               srA, scA, s1A, s2A, swgA, swxA,
               srB, scB, s1B, s2B, swgB, swxB):
    wid = lax.axis_index("s") * _NC + lax.axis_index("c")
    base0 = wid * _EW

    par = [(idxrA, idxcA, bufA, srA, scA, s1A, s2A, swgA, swxA),
           (idxrB, idxcB, bufB, srB, scB, s1B, s2B, swgB, swxB)]
    writes = [None, None]
    pend = None

    for i in range(_NCHUNK + 1):
        if i < _NCHUNK:
            p = i % 2
            idxr, idxc, buf, sr, sc, s1, s2, swg, swx = par[p]
            base = base0 + i * _C
            if writes[p] is not None:
                w1, w2 = writes[p]
                w1.wait()
                w2.wait()
                writes[p] = None
            lr = pltpu.async_copy(row_h.at[pl.ds(base, _C)], idxr, sr)
            lc = pltpu.async_copy(col_h.at[pl.ds(base, _C)], idxc, sc)
            lr.wait()
            c1 = pltpu.async_copy(t1_h.at[idxr], buf, s1)
        if pend is not None:
            pc2, pbuf, pbase, pswg, pswx, pp = pend
            pc2.wait()
            w1 = pltpu.async_copy(pbuf.at[:, pl.ds(0, _HID)],
                                  g_h.at[pl.ds(pbase, _C)], pswg)
            w2 = pltpu.async_copy(pbuf.at[:, pl.ds(_HID, _XW)],
                                  xd_h.at[pl.ds(pbase, _C), pl.ds(0, _XW)],
                                  pswx)
            writes[pp] = (w1, w2)
            pend = None
        if i < _NCHUNK:
            c1.wait()
            lc.wait()
            c2 = pltpu.async_copy(t2_h.at[idxc], buf, s2, add=True)
            pend = (c2, buf, base, swg, swx, p)

    for p in range(2):
        if writes[p] is not None:
            w1, w2 = writes[p]
            w1.wait()
            w2.wait()


@functools.partial(
    pl.kernel,
    out_type=(
        jax.ShapeDtypeStruct((_NC, _NPAD, _HID), jnp.float32),
        jax.ShapeDtypeStruct((_NC, _NPAD, _XW), jnp.float32),
    ),
    mesh=plsc.VectorSubcoreMesh(core_axis_name="c", subcore_axis_name="s"),
    scratch_types=[
        pltpu.VMEM((_CS,), jnp.int32),
        pltpu.VMEM((_CS, _HID), jnp.float32),
        pltpu.VMEM((_CS, _XW), jnp.float32),
        pltpu.VMEM((_CS,), jnp.int32),
        pltpu.VMEM((_CS, _HID), jnp.float32),
        pltpu.VMEM((_CS, _XW), jnp.float32),
        pltpu.VMEM_SHARED((_NPAD, _HID), jnp.float32),
        pltpu.VMEM_SHARED((_NPAD, _XW), jnp.float32),
        pltpu.SemaphoreType.DMA,
        pltpu.SemaphoreType.DMA,
        pltpu.SemaphoreType.DMA,
        pltpu.SemaphoreType.DMA,
        pltpu.SemaphoreType.DMA,
        pltpu.SemaphoreType.DMA,
        pltpu.SemaphoreType.DMA,
        pltpu.SemaphoreType.DMA,
        pltpu.SemaphoreType.DMA,
        pltpu.SemaphoreType.DMA,
    ],
    compiler_params=pltpu.CompilerParams(use_tc_tiling_on_sc=False),
    name="egnn_sc_scatter",
)
def _sc_scatter(ma_h, cu_h, row_h, z128_h, z16_h, hp_h, xp_h,
                idxA, bufmA, bufcA, idxB, bufmB, bufcB, hacc, xacc,
                liA, lmA, lcA, ahA, axA, liB, lmB, lcB, ahB, axB):
    cid = lax.axis_index("c")
    sid = lax.axis_index("s")
    wid = sid * _NC + cid
    base0 = wid * _EW
    rbase = sid * _RPT

    pltpu.sync_copy(z128_h, hacc.at[pl.ds(rbase, _RPT)])
    pltpu.sync_copy(z16_h, xacc.at[pl.ds(rbase, _RPT)])
    plsc.subcore_barrier()

    par = [(idxA, bufmA, bufcA, liA, lmA, lcA, ahA, axA),
           (idxB, bufmB, bufcB, liB, lmB, lcB, ahB, axB)]
    adds = [None, None]
    pend = None

    for i in range(_NCHUNKS + 1):
        if i < _NCHUNKS:
            p = i % 2
            idx, bufm, bufc, li, lm, lc, ah, ax = par[p]
            base = base0 + i * _CS
            if adds[p] is not None:
                a1, a2 = adds[p]
                a1.wait()
                a2.wait()
                adds[p] = None
            l1 = pltpu.async_copy(row_h.at[pl.ds(base, _CS)], idx, li)
            l2 = pltpu.async_copy(ma_h.at[pl.ds(base, _CS)], bufm, lm)
            l3 = pltpu.async_copy(cu_h.at[pl.ds(base, _CS), pl.ds(0, _XW)],
                                  bufc, lc)
        if pend is not None:
            pl1, pl2, pl3, pp = pend
            pl1.wait()
            pl2.wait()
            pl3.wait()
            pidx, pbufm, pbufc, _, _, _, pah, pax = par[pp]
            a1 = pltpu.async_copy(pbufm, hacc.at[pidx], pah, add=True)
            a2 = pltpu.async_copy(pbufc, xacc.at[pidx], pax, add=True)
            adds[pp] = (a1, a2)
            pend = None
        if i < _NCHUNKS:
            pend = (l1, l2, l3, p)

    for p in range(2):
        if adds[p] is not None:
            a1, a2 = adds[p]
            a1.wait()
            a2.wait()
    plsc.subcore_barrier()

    pltpu.sync_copy(hacc.at[pl.ds(rbase, _RPT)], hp_h.at[cid, pl.ds(rbase, _RPT)])
    pltpu.sync_copy(xacc.at[pl.ds(rbase, _RPT)], xp_h.at[cid, pl.ds(rbase, _RPT)])


def _tc_prologue(hpad, t11, neT, neb, tw1T, tb1, tw2T, tb2, whrT, whcT):
    def body(t_ref, h_ref, neT_ref, neb_ref, tw1T_ref, tb1_ref, tw2T_ref,
             tb2_ref, whrT_ref, whcT_ref, h0_ref, hr_ref, hc_ref):
        tval = t_ref[0, 0]
        half = _TDIM // 2
        lane_i = lax.broadcasted_iota(jnp.int32, (1, _TDIM), 1)
        lane = lane_i.astype(jnp.float32)
        k = jnp.where(lane < half, lane, lane - half)
        freq = jnp.exp(k * (-(np.log(10000.0) / (half - 1))))
        arg = tval * freq
        te0 = jnp.where(lane < half, jnp.sin(arg), jnp.cos(arg))
        te1 = _silu(jnp.dot(te0, tw1T_ref[...], preferred_element_type=jnp.float32)
                    + tb1_ref[...])
        te2 = (jnp.dot(te1, tw2T_ref[...], preferred_element_type=jnp.float32)
               + tb2_ref[...])
        h0 = (jnp.dot(h_ref[...], neT_ref[...], preferred_element_type=jnp.float32)
              + neb_ref[...] + te2)
        h0_ref[...] = h0
        hr_ref[...] = jnp.dot(h0, whrT_ref[...], preferred_element_type=jnp.float32)
        hc_ref[...] = jnp.dot(h0, whcT_ref[...], preferred_element_type=jnp.float32)

    full = lambda shape: pl.BlockSpec(shape, lambda i: (0, 0))
    return pl.pallas_call(
        body,
        grid=(_NBN,),
        in_specs=[
            pl.BlockSpec((1, 1), lambda i: (0, 0), memory_space=pltpu.SMEM),
            pl.BlockSpec((_BN, _HID), lambda i: (i, 0)),
            full((_HID, _HID)), full((1, _HID)),
            full((_TDIM, _HID)), full((1, _HID)),
            full((_HID, _HID)), full((1, _HID)),
            full((_HID, _HID)), full((_HID, _HID)),
        ],
        out_specs=[
            pl.BlockSpec((_BN, _HID), lambda i: (i, 0)),
            pl.BlockSpec((_BN, _HID), lambda i: (i, 0)),
            pl.BlockSpec((_BN, _HID), lambda i: (i, 0)),
        ],
        out_shape=[
            jax.ShapeDtypeStruct((_NPAD, _HID), jnp.float32),
            jax.ShapeDtypeStruct((_NPAD, _HID), jnp.float32),
            jax.ShapeDtypeStruct((_NPAD, _HID), jnp.float32),
        ],
        name="egnn_tc_prologue",
    )(t11, hpad, neT, neb, tw1T, tb1, tw2T, tb2, whrT, whcT)


def _tc_edge(g, xdr, ea, w_r, w_eaT, eb1, ew2T, eb2, aw, ab11,
             cw1T, cb1, cw2):
    def body(ab_ref, g_ref, xdr_ref, ea_ref, wr_ref,
             weaT_ref, eb1_ref, ew2T_ref, eb2_ref, aw_ref, cw1T_ref, cb1_ref,
             cw2_ref, ma_ref, cu_ref):
        g = g_ref[...]
        lane = lax.broadcasted_iota(jnp.int32, (_BE, _HID), 1)
        xd = jnp.where(lane < _XW, xdr_ref[...], 0.0)
        radial = jnp.sum(xd * xd, axis=-1, keepdims=True)
        pre = (g + radial * wr_ref[...]
               + jnp.dot(ea_ref[...], weaT_ref[...],
                         preferred_element_type=jnp.float32)
               + eb1_ref[...])
        m = _silu(pre)
        m = _silu(jnp.dot(m, ew2T_ref[...], preferred_element_type=jnp.float32)
                  + eb2_ref[...])
        att = _sigmoid(jnp.sum(m * aw_ref[...], axis=-1, keepdims=True)
                       + ab_ref[0, 0])
        m = m * att
        c1 = _silu(jnp.dot(m, cw1T_ref[...], preferred_element_type=jnp.float32)
                   + cb1_ref[...])
        cws = jnp.sum(c1 * cw2_ref[...], axis=-1, keepdims=True)
        cu_ref[...] = xd * (cws / jnp.sqrt(radial + 1e-08))
        ma_ref[...] = m

    full = lambda shape: pl.BlockSpec(shape, lambda i: (0, 0))
    eb = lambda w: pl.BlockSpec((_BE, w), lambda i: (i, 0))
    return pl.pallas_call(
        body,
        grid=(_EP // _BE,),
        in_specs=[
            pl.BlockSpec((1, 1), lambda i: (0, 0), memory_space=pltpu.SMEM),
            eb(_HID), eb(_HID), eb(_EDIM),
            full((1, _HID)), full((_EDIM, _HID)), full((1, _HID)),
            full((_HID, _HID)), full((1, _HID)), full((1, _HID)),
            full((_HID, _HID)), full((1, _HID)), full((1, _HID)),
        ],
        out_specs=[eb(_HID), eb(_HID)],
        out_shape=[
            jax.ShapeDtypeStruct((_EP, _HID), jnp.float32),
            jax.ShapeDtypeStruct((_EP, _HID), jnp.float32),
        ],
        name="egnn_tc_edge",
    )(ab11, g, xdr, ea, w_r, w_eaT, eb1, ew2T, eb2, aw,
      cw1T, cb1, cw2)


def _tc_node(h, hps, nw1hT, nw1mT, nb1, nw2T, nb2, ln_g, ln_b,
             whrT, whcT):
    nparts = len(hps)

    def body(*refs):
        h_ref = refs[0]
        hp_refs = refs[1:1 + nparts]
        (nw1hT_ref, nw1mT_ref, nb1_ref, nw2T_ref, nb2_ref, g_ref, b_ref,
         whrT_ref, whcT_ref, hn_ref, hr_ref, hc_ref) = refs[1 + nparts:]
        hv = h_ref[...]
        mi = hp_refs[0][...]
        for r in hp_refs[1:]:
            mi = mi + r[...]
        a = _silu(jnp.dot(hv, nw1hT_ref[...], preferred_element_type=jnp.float32)
                  + jnp.dot(mi, nw1mT_ref[...], preferred_element_type=jnp.float32)
                  + nb1_ref[...])
        hn = hv + jnp.dot(a, nw2T_ref[...], preferred_element_type=jnp.float32) \
            + nb2_ref[...]
        mu = jnp.mean(hn, axis=-1, keepdims=True)
        var = jnp.mean((hn - mu) * (hn - mu), axis=-1, keepdims=True)
        hn = (hn - mu) / jnp.sqrt(var + 1e-05) * g_ref[...] + b_ref[...]
        hn_ref[...] = hn
        hr_ref[...] = jnp.dot(hn, whrT_ref[...], preferred_element_type=jnp.float32)
        hc_ref[...] = jnp.dot(hn, whcT_ref[...], preferred_element_type=jnp.float32)

    full = lambda shape: pl.BlockSpec(shape, lambda i: (0, 0))
    nb = pl.BlockSpec((_BN, _HID), lambda i: (i, 0))
    return pl.pallas_call(
        body,
        grid=(_NBN,),
        in_specs=[nb] * (1 + nparts) + [
            full((_HID, _HID)), full((_HID, _HID)), full((1, _HID)),
            full((_HID, _HID)), full((1, _HID)), full((1, _HID)),
            full((1, _HID)), full((_HID, _HID)), full((_HID, _HID)),
        ],
        out_specs=[nb, nb, nb],
        out_shape=[
            jax.ShapeDtypeStruct((_NPAD, _HID), jnp.float32),
            jax.ShapeDtypeStruct((_NPAD, _HID), jnp.float32),
            jax.ShapeDtypeStruct((_NPAD, _HID), jnp.float32),
        ],
        name="egnn_tc_node",
    )(h, *hps, nw1hT, nw1mT, nb1, nw2T, nb2, ln_g, ln_b, whrT, whcT)


def _tc_epilogue(h, om1T, omb1, om2T, omb2, chT16, chb16):
    def body(h_ref, om1T_ref, omb1_ref, om2T_ref, omb2_ref, chT_ref, chb_ref,
             ho_ref, xd_ref):
        hv = h_ref[...]
        a = _silu(jnp.dot(hv, om1T_ref[...], preferred_element_type=jnp.float32)
                  + omb1_ref[...])
        ho_ref[...] = jnp.dot(a, om2T_ref[...], preferred_element_type=jnp.float32) \
            + omb2_ref[...]
        xd_ref[...] = jnp.dot(hv, chT_ref[...], preferred_element_type=jnp.float32) \
            + chb_ref[...]

    full = lambda shape: pl.BlockSpec(shape, lambda i: (0, 0))
    nb = pl.BlockSpec((_BN, _HID), lambda i: (i, 0))
    return pl.pallas_call(
        body,
        grid=(_NBN,),
        in_specs=[
            nb,
            full((_HID, _HID)), full((1, _HID)),
            full((_HID, _HID)), full((1, _HID)),
            full((_HID, _XW)), full((1, _XW)),
        ],
        out_specs=[nb, pl.BlockSpec((_BN, _XW), lambda i: (i, 0))],
        out_shape=[
            jax.ShapeDtypeStruct((_NPAD, _HID), jnp.float32),
            jax.ShapeDtypeStruct((_NPAD, _XW), jnp.float32),
        ],
        name="egnn_tc_epilogue",
    )(h, om1T, omb1, om2T, omb2, chT16, chb16)


def kernel(h, x, edge_index, t, edge_attr, params):
    p = params
    rows = [edge_index[0, k * _EP:(k + 1) * _EP] for k in range(_K)]
    cols = [edge_index[1, k * _EP:(k + 1) * _EP] for k in range(_K)]
    eas = [edge_attr[k * _EP:(k + 1) * _EP] for k in range(_K)]

    hpad = jnp.zeros((_NPAD, _HID), jnp.float32).at[:_N].set(h)
    xpad = jnp.zeros((_NPAD, _XW), jnp.float32).at[:_N, :3].set(x)
    x_init = xpad
    t11 = t.reshape(1, 1)
    z128 = jnp.zeros((_RPT, _HID), jnp.float32)
    z16 = jnp.zeros((_RPT, _XW), jnp.float32)

    whrT = [p['ew1'][i][:, :_HID].T for i in range(_L)]
    whcT = [p['ew1'][i][:, _HID:2 * _HID].T for i in range(_L)]
    w_r = [p['ew1'][i][:, 2 * _HID].reshape(1, _HID) for i in range(_L)]
    w_eaT = [p['ew1'][i][:, 2 * _HID + 1:].T for i in range(_L)]

    hcur, hrp, hcp = _tc_prologue(
        hpad, t11,
        p['ne_w'].T, p['ne_b'].reshape(1, _HID),
        p['te_w1'].T, p['te_b1'].reshape(1, _HID),
        p['te_w2'].T, p['te_b2'].reshape(1, _HID),
        whrT[0], whcT[0])

    xcur = xpad
    for i in range(_L):
        hparts = []
        xparts = []
        t1 = jnp.concatenate([hrp, xcur], axis=1)
        t2 = jnp.concatenate([hcp, -xcur], axis=1)
        for k in range(_K):
            g, xdr = _sc_gather(t1, t2, rows[k], cols[k])
            ma, cu = _tc_edge(
                g, xdr, eas[k],
                w_r[i], w_eaT[i], p['eb1'][i].reshape(1, _HID),
                p['ew2'][i].T, p['eb2'][i].reshape(1, _HID),
                p['aw'][i], p['ab'][i].reshape(1, 1),
                p['cw1'][i].T, p['cb1'][i].reshape(1, _HID), p['cw2'][i])
            hp, xp = _sc_scatter(ma, cu, rows[k], z128, z16)
            hparts.extend([hp[0], hp[1]])
            xparts.extend([xp[0], xp[1]])
        j = min(i + 1, _L - 1)
        hcur, hrp, hcp = _tc_node(
            hcur, hparts,
            p['nw1'][i][:, :_HID].T, p['nw1'][i][:, _HID:].T,
            p['nb1'][i].reshape(1, _HID),
            p['nw2'][i].T, p['nb2'][i].reshape(1, _HID),
            p['ln_g'][i].reshape(1, _HID), p['ln_b'][i].reshape(1, _HID),
            whrT[j], whcT[j])
        for xp_part in xparts:
            xcur = xcur + xp_part

    chT16 = jnp.zeros((_HID, _XW), jnp.float32).at[:, :3].set(p['ch_w'].T)
    chb16 = jnp.zeros((1, _XW), jnp.float32).at[0, :3].set(p['ch_b'])
    hout, xd = _tc_epilogue(
        hcur,
        p['om_w1'].T, p['om_b1'].reshape(1, _HID),
        p['om_w2'].T, p['om_b2'].reshape(1, _HID),
        chT16, chb16)

    x_out = (xcur - x_init)[:_N, :3] + xd[:_N, :3]
    return (hout[:_N], x_out)


# speedup vs baseline: 2.7860x; 1.0167x over previous
import functools

import numpy as np
import jax
import jax.numpy as jnp
from jax import lax
from jax.experimental import pallas as pl
from jax.experimental.pallas import tpu as pltpu
from jax.experimental.pallas import tpu_sc as plsc

_N = 10000
_NPAD = 10240
_E = 320000
_HID = 128
_EDIM = 16
_TDIM = 64
_L = 4
_XW = 16

_NC = 2
_NS = 16
_NW = _NC * _NS

_K = 2
_EP = _E // _K
_EW = _EP // _NW
_C = 200
_NCHUNK = _EW // _C
_CS = 40
_NCHUNKS = _EW // _CS
_RPT = _NPAD // _NS

_BE = 3200
_BN = 512
_NBN = _NPAD // _BN


def _silu(v):
    return v * (1.0 / (1.0 + jnp.exp(-v)))


def _sigmoid(v):
    return 1.0 / (1.0 + jnp.exp(-v))


_TW = _HID + _XW


@functools.partial(
    pl.kernel,
    out_type=(
        jax.ShapeDtypeStruct((_EP, _HID), jnp.float32),
        jax.ShapeDtypeStruct((_EP, _HID), jnp.float32),
    ),
    mesh=plsc.VectorSubcoreMesh(core_axis_name="c", subcore_axis_name="s"),
    scratch_types=[
        pltpu.VMEM((_C,), jnp.int32),
        pltpu.VMEM((_C,), jnp.int32),
        pltpu.VMEM((_C, _TW), jnp.float32),
        pltpu.VMEM((_C,), jnp.int32),
        pltpu.VMEM((_C,), jnp.int32),
        pltpu.VMEM((_C, _TW), jnp.float32),
        pltpu.SemaphoreType.DMA,
        pltpu.SemaphoreType.DMA,
        pltpu.SemaphoreType.DMA,
        pltpu.SemaphoreType.DMA,
        pltpu.SemaphoreType.DMA,
        pltpu.SemaphoreType.DMA,
        pltpu.SemaphoreType.DMA,
        pltpu.SemaphoreType.DMA,
        pltpu.SemaphoreType.DMA,
        pltpu.SemaphoreType.DMA,
        pltpu.SemaphoreType.DMA,
        pltpu.SemaphoreType.DMA,
    ],
    compiler_params=pltpu.CompilerParams(use_tc_tiling_on_sc=False),
    name="egnn_sc_gather",
)
def _sc_gather(t1_h, t2_h, row_h, col_h, g_h, xd_h,
               idxrA, idxcA, bufA, idxrB, idxcB, bufB,
               srA, scA, s1A, s2A, swgA, swxA,
               srB, scB, s1B, s2B, swgB, swxB):
    wid = lax.axis_index("s") * _NC + lax.axis_index("c")
    base0 = wid * _EW

    par = [(idxrA, idxcA, bufA, srA, scA, s1A, s2A, swgA, swxA),
           (idxrB, idxcB, bufB, srB, scB, s1B, s2B, swgB, swxB)]
    writes = [None, None]
    pend = None

    for i in range(_NCHUNK + 1):
        if i < _NCHUNK:
            p = i % 2
            idxr, idxc, buf, sr, sc, s1, s2, swg, swx = par[p]
            base = base0 + i * _C
            if writes[p] is not None:
                w1, w2 = writes[p]
                w1.wait()
                w2.wait()
                writes[p] = None
            lr = pltpu.async_copy(row_h.at[pl.ds(base, _C)], idxr, sr)
            lc = pltpu.async_copy(col_h.at[pl.ds(base, _C)], idxc, sc)
            lr.wait()
            c1 = pltpu.async_copy(t1_h.at[idxr], buf, s1)
        if pend is not None:
            pc2, pbuf, pbase, pswg, pswx, pp = pend
            pc2.wait()
            w1 = pltpu.async_copy(pbuf.at[:, pl.ds(0, _HID)],
                                  g_h.at[pl.ds(pbase, _C)], pswg)
            w2 = pltpu.async_copy(pbuf.at[:, pl.ds(_HID, _XW)],
                                  xd_h.at[pl.ds(pbase, _C), pl.ds(0, _XW)],
                                  pswx)
            writes[pp] = (w1, w2)
            pend = None
        if i < _NCHUNK:
            c1.wait()
            lc.wait()
            c2 = pltpu.async_copy(t2_h.at[idxc], buf, s2, add=True)
            pend = (c2, buf, base, swg, swx, p)

    for p in range(2):
        if writes[p] is not None:
            w1, w2 = writes[p]
            w1.wait()
            w2.wait()


@functools.partial(
    pl.kernel,
    out_type=(
        jax.ShapeDtypeStruct((_NC, _NPAD, _HID), jnp.float32),
        jax.ShapeDtypeStruct((_NC, _NPAD, _XW), jnp.float32),
    ),
    mesh=plsc.VectorSubcoreMesh(core_axis_name="c", subcore_axis_name="s"),
    scratch_types=[
        pltpu.VMEM((_CS,), jnp.int32),
        pltpu.VMEM((_CS, _HID), jnp.float32),
        pltpu.VMEM((_CS, _XW), jnp.float32),
        pltpu.VMEM((_CS,), jnp.int32),
        pltpu.VMEM((_CS, _HID), jnp.float32),
        pltpu.VMEM((_CS, _XW), jnp.float32),
        pltpu.VMEM_SHARED((_NPAD, _HID), jnp.float32),
        pltpu.VMEM_SHARED((_NPAD, _XW), jnp.float32),
        pltpu.SemaphoreType.DMA,
        pltpu.SemaphoreType.DMA,
        pltpu.SemaphoreType.DMA,
        pltpu.SemaphoreType.DMA,
        pltpu.SemaphoreType.DMA,
        pltpu.SemaphoreType.DMA,
        pltpu.SemaphoreType.DMA,
        pltpu.SemaphoreType.DMA,
        pltpu.SemaphoreType.DMA,
        pltpu.SemaphoreType.DMA,
    ],
    compiler_params=pltpu.CompilerParams(use_tc_tiling_on_sc=False),
    name="egnn_sc_scatter",
)
def _sc_scatter(ma_h, cu_h, row_h, z128_h, z16_h, hp_h, xp_h,
                idxA, bufmA, bufcA, idxB, bufmB, bufcB, hacc, xacc,
                liA, lmA, lcA, ahA, axA, liB, lmB, lcB, ahB, axB):
    cid = lax.axis_index("c")
    sid = lax.axis_index("s")
    wid = sid * _NC + cid
    base0 = wid * _EW
    rbase = sid * _RPT

    pltpu.sync_copy(z128_h, hacc.at[pl.ds(rbase, _RPT)])
    pltpu.sync_copy(z16_h, xacc.at[pl.ds(rbase, _RPT)])
    plsc.subcore_barrier()

    par = [(idxA, bufmA, bufcA, liA, lmA, lcA, ahA, axA),
           (idxB, bufmB, bufcB, liB, lmB, lcB, ahB, axB)]
    adds = [None, None]
    pend = None

    for i in range(_NCHUNKS + 1):
        if i < _NCHUNKS:
            p = i % 2
            idx, bufm, bufc, li, lm, lc, ah, ax = par[p]
            base = base0 + i * _CS
            if adds[p] is not None:
                a1, a2 = adds[p]
                a1.wait()
                a2.wait()
                adds[p] = None
            l1 = pltpu.async_copy(row_h.at[pl.ds(base, _CS)], idx, li)
            l2 = pltpu.async_copy(ma_h.at[pl.ds(base, _CS)], bufm, lm)
            l3 = pltpu.async_copy(cu_h.at[pl.ds(base, _CS), pl.ds(0, _XW)],
                                  bufc, lc)
        if pend is not None:
            pl1, pl2, pl3, pp = pend
            pl1.wait()
            pl2.wait()
            pl3.wait()
            pidx, pbufm, pbufc, _, _, _, pah, pax = par[pp]
            a1 = pltpu.async_copy(pbufm, hacc.at[pidx], pah, add=True)
            a2 = pltpu.async_copy(pbufc, xacc.at[pidx], pax, add=True)
            adds[pp] = (a1, a2)
            pend = None
        if i < _NCHUNKS:
            pend = (l1, l2, l3, p)

    for p in range(2):
        if adds[p] is not None:
            a1, a2 = adds[p]
            a1.wait()
            a2.wait()
    plsc.subcore_barrier()

    pltpu.sync_copy(hacc.at[pl.ds(rbase, _RPT)], hp_h.at[cid, pl.ds(rbase, _RPT)])
    pltpu.sync_copy(xacc.at[pl.ds(rbase, _RPT)], xp_h.at[cid, pl.ds(rbase, _RPT)])


def _tc_prologue(hpad, t11, neT, neb, tw1T, tb1, tw2T, tb2, whrT, whcT):
    def body(t_ref, h_ref, neT_ref, neb_ref, tw1T_ref, tb1_ref, tw2T_ref,
             tb2_ref, whrT_ref, whcT_ref, h0_ref, hr_ref, hc_ref):
        tval = t_ref[0, 0]
        half = _TDIM // 2
        lane_i = lax.broadcasted_iota(jnp.int32, (1, _TDIM), 1)
        lane = lane_i.astype(jnp.float32)
        k = jnp.where(lane < half, lane, lane - half)
        freq = jnp.exp(k * (-(np.log(10000.0) / (half - 1))))
        arg = tval * freq
        te0 = jnp.where(lane < half, jnp.sin(arg), jnp.cos(arg))
        te1 = _silu(jnp.dot(te0, tw1T_ref[...], preferred_element_type=jnp.float32)
                    + tb1_ref[...])
        te2 = (jnp.dot(te1, tw2T_ref[...], preferred_element_type=jnp.float32)
               + tb2_ref[...])
        h0 = (jnp.dot(h_ref[...], neT_ref[...], preferred_element_type=jnp.float32)
              + neb_ref[...] + te2)
        h0_ref[...] = h0
        hr_ref[...] = jnp.dot(h0, whrT_ref[...], preferred_element_type=jnp.float32)
        hc_ref[...] = jnp.dot(h0, whcT_ref[...], preferred_element_type=jnp.float32)

    full = lambda shape: pl.BlockSpec(shape, lambda i: (0, 0))
    return pl.pallas_call(
        body,
        grid=(_NBN,),
        in_specs=[
            pl.BlockSpec((1, 1), lambda i: (0, 0), memory_space=pltpu.SMEM),
            pl.BlockSpec((_BN, _HID), lambda i: (i, 0)),
            full((_HID, _HID)), full((1, _HID)),
            full((_TDIM, _HID)), full((1, _HID)),
            full((_HID, _HID)), full((1, _HID)),
            full((_HID, _HID)), full((_HID, _HID)),
        ],
        out_specs=[
            pl.BlockSpec((_BN, _HID), lambda i: (i, 0)),
            pl.BlockSpec((_BN, _HID), lambda i: (i, 0)),
            pl.BlockSpec((_BN, _HID), lambda i: (i, 0)),
        ],
        out_shape=[
            jax.ShapeDtypeStruct((_NPAD, _HID), jnp.float32),
            jax.ShapeDtypeStruct((_NPAD, _HID), jnp.float32),
            jax.ShapeDtypeStruct((_NPAD, _HID), jnp.float32),
        ],
        name="egnn_tc_prologue",
    )(t11, hpad, neT, neb, tw1T, tb1, tw2T, tb2, whrT, whcT)


def _tc_edge(g, xdr, ea, w_r, w_eaT, eb1, ew2T, eb2, aw, ab11,
             cw1T, cb1, cw2):
    def body(ab_ref, g_ref, xdr_ref, ea_ref, wr_ref,
             weaT_ref, eb1_ref, ew2T_ref, eb2_ref, aw_ref, cw1T_ref, cb1_ref,
             cw2_ref, ma_ref, cu_ref):
        g = g_ref[...]
        lane = lax.broadcasted_iota(jnp.int32, (_BE, _HID), 1)
        xd = jnp.where(lane < _XW, xdr_ref[...], 0.0)
        radial = jnp.sum(xd * xd, axis=-1, keepdims=True)
        pre = (g + radial * wr_ref[...]
               + jnp.dot(ea_ref[...], weaT_ref[...],
                         preferred_element_type=jnp.float32)
               + eb1_ref[...])
        m = _silu(pre)
        m = _silu(jnp.dot(m, ew2T_ref[...], preferred_element_type=jnp.float32)
                  + eb2_ref[...])
        att = _sigmoid(jnp.sum(m * aw_ref[...], axis=-1, keepdims=True)
                       + ab_ref[0, 0])
        m = m * att
        c1 = _silu(jnp.dot(m, cw1T_ref[...], preferred_element_type=jnp.float32)
                   + cb1_ref[...])
        cws = jnp.sum(c1 * cw2_ref[...], axis=-1, keepdims=True)
        cu_ref[...] = xd * (cws / jnp.sqrt(radial + 1e-08))
        ma_ref[...] = m

    full = lambda shape: pl.BlockSpec(shape, lambda i: (0, 0))
    eb = lambda w: pl.BlockSpec((_BE, w), lambda i: (i, 0))
    return pl.pallas_call(
        body,
        grid=(_EP // _BE,),
        in_specs=[
            pl.BlockSpec((1, 1), lambda i: (0, 0), memory_space=pltpu.SMEM),
            eb(_HID), eb(_HID), eb(_EDIM),
            full((1, _HID)), full((_EDIM, _HID)), full((1, _HID)),
            full((_HID, _HID)), full((1, _HID)), full((1, _HID)),
            full((_HID, _HID)), full((1, _HID)), full((1, _HID)),
        ],
        out_specs=[eb(_HID), eb(_HID)],
        out_shape=[
            jax.ShapeDtypeStruct((_EP, _HID), jnp.float32),
            jax.ShapeDtypeStruct((_EP, _HID), jnp.float32),
        ],
        name="egnn_tc_edge",
    )(ab11, g, xdr, ea, w_r, w_eaT, eb1, ew2T, eb2, aw,
      cw1T, cb1, cw2)


def _tc_node(h, hps, nw1hT, nw1mT, nb1, nw2T, nb2, ln_g, ln_b,
             whrT, whcT):
    nparts = len(hps)

    def body(*refs):
        h_ref = refs[0]
        hp_refs = refs[1:1 + nparts]
        (nw1hT_ref, nw1mT_ref, nb1_ref, nw2T_ref, nb2_ref, g_ref, b_ref,
         whrT_ref, whcT_ref, hn_ref, hr_ref, hc_ref) = refs[1 + nparts:]
        hv = h_ref[...]
        mi = hp_refs[0][0] + hp_refs[0][1]
        for r in hp_refs[1:]:
            mi = mi + r[0] + r[1]
        a = _silu(jnp.dot(hv, nw1hT_ref[...], preferred_element_type=jnp.float32)
                  + jnp.dot(mi, nw1mT_ref[...], preferred_element_type=jnp.float32)
                  + nb1_ref[...])
        hn = hv + jnp.dot(a, nw2T_ref[...], preferred_element_type=jnp.float32) \
            + nb2_ref[...]
        mu = jnp.mean(hn, axis=-1, keepdims=True)
        var = jnp.mean((hn - mu) * (hn - mu), axis=-1, keepdims=True)
        hn = (hn - mu) / jnp.sqrt(var + 1e-05) * g_ref[...] + b_ref[...]
        hn_ref[...] = hn
        hr_ref[...] = jnp.dot(hn, whrT_ref[...], preferred_element_type=jnp.float32)
        hc_ref[...] = jnp.dot(hn, whcT_ref[...], preferred_element_type=jnp.float32)

    full = lambda shape: pl.BlockSpec(shape, lambda i: (0, 0))
    nb = pl.BlockSpec((_BN, _HID), lambda i: (i, 0))
    pb = pl.BlockSpec((_NC, _BN, _HID), lambda i: (0, i, 0))
    return pl.pallas_call(
        body,
        grid=(_NBN,),
        in_specs=[nb] + [pb] * nparts + [
            full((_HID, _HID)), full((_HID, _HID)), full((1, _HID)),
            full((_HID, _HID)), full((1, _HID)), full((1, _HID)),
            full((1, _HID)), full((_HID, _HID)), full((_HID, _HID)),
        ],
        out_specs=[nb, nb, nb],
        out_shape=[
            jax.ShapeDtypeStruct((_NPAD, _HID), jnp.float32),
            jax.ShapeDtypeStruct((_NPAD, _HID), jnp.float32),
            jax.ShapeDtypeStruct((_NPAD, _HID), jnp.float32),
        ],
        name="egnn_tc_node",
    )(h, *hps, nw1hT, nw1mT, nb1, nw2T, nb2, ln_g, ln_b, whrT, whcT)


def _tc_epilogue(h, om1T, omb1, om2T, omb2, chT16, chb16):
    def body(h_ref, om1T_ref, omb1_ref, om2T_ref, omb2_ref, chT_ref, chb_ref,
             ho_ref, xd_ref):
        hv = h_ref[...]
        a = _silu(jnp.dot(hv, om1T_ref[...], preferred_element_type=jnp.float32)
                  + omb1_ref[...])
        ho_ref[...] = jnp.dot(a, om2T_ref[...], preferred_element_type=jnp.float32) \
            + omb2_ref[...]
        xd_ref[...] = jnp.dot(hv, chT_ref[...], preferred_element_type=jnp.float32) \
            + chb_ref[...]

    full = lambda shape: pl.BlockSpec(shape, lambda i: (0, 0))
    nb = pl.BlockSpec((_BN, _HID), lambda i: (i, 0))
    return pl.pallas_call(
        body,
        grid=(_NBN,),
        in_specs=[
            nb,
            full((_HID, _HID)), full((1, _HID)),
            full((_HID, _HID)), full((1, _HID)),
            full((_HID, _XW)), full((1, _XW)),
        ],
        out_specs=[nb, pl.BlockSpec((_BN, _XW), lambda i: (i, 0))],
        out_shape=[
            jax.ShapeDtypeStruct((_NPAD, _HID), jnp.float32),
            jax.ShapeDtypeStruct((_NPAD, _XW), jnp.float32),
        ],
        name="egnn_tc_epilogue",
    )(h, om1T, omb1, om2T, omb2, chT16, chb16)


def kernel(h, x, edge_index, t, edge_attr, params):
    p = params
    rows = [edge_index[0, k * _EP:(k + 1) * _EP] for k in range(_K)]
    cols = [edge_index[1, k * _EP:(k + 1) * _EP] for k in range(_K)]
    eas = [edge_attr[k * _EP:(k + 1) * _EP] for k in range(_K)]

    hpad = jnp.zeros((_NPAD, _HID), jnp.float32).at[:_N].set(h)
    xpad = jnp.zeros((_NPAD, _XW), jnp.float32).at[:_N, :3].set(x)
    x_init = xpad
    t11 = t.reshape(1, 1)
    z128 = jnp.zeros((_RPT, _HID), jnp.float32)
    z16 = jnp.zeros((_RPT, _XW), jnp.float32)

    whrT = [p['ew1'][i][:, :_HID].T for i in range(_L)]
    whcT = [p['ew1'][i][:, _HID:2 * _HID].T for i in range(_L)]
    w_r = [p['ew1'][i][:, 2 * _HID].reshape(1, _HID) for i in range(_L)]
    w_eaT = [p['ew1'][i][:, 2 * _HID + 1:].T for i in range(_L)]

    hcur, hrp, hcp = _tc_prologue(
        hpad, t11,
        p['ne_w'].T, p['ne_b'].reshape(1, _HID),
        p['te_w1'].T, p['te_b1'].reshape(1, _HID),
        p['te_w2'].T, p['te_b2'].reshape(1, _HID),
        whrT[0], whcT[0])

    xcur = xpad
    for i in range(_L):
        hparts = []
        xparts = []
        t1 = jnp.concatenate([hrp, xcur], axis=1)
        t2 = jnp.concatenate([hcp, -xcur], axis=1)
        for k in range(_K):
            g, xdr = _sc_gather(t1, t2, rows[k], cols[k])
            ma, cu = _tc_edge(
                g, xdr, eas[k],
                w_r[i], w_eaT[i], p['eb1'][i].reshape(1, _HID),
                p['ew2'][i].T, p['eb2'][i].reshape(1, _HID),
                p['aw'][i], p['ab'][i].reshape(1, 1),
                p['cw1'][i].T, p['cb1'][i].reshape(1, _HID), p['cw2'][i])
            hp, xp = _sc_scatter(ma, cu, rows[k], z128, z16)
            hparts.append(hp)
            xparts.extend([xp[0], xp[1]])
        j = min(i + 1, _L - 1)
        hcur, hrp, hcp = _tc_node(
            hcur, hparts,
            p['nw1'][i][:, :_HID].T, p['nw1'][i][:, _HID:].T,
            p['nb1'][i].reshape(1, _HID),
            p['nw2'][i].T, p['nb2'][i].reshape(1, _HID),
            p['ln_g'][i].reshape(1, _HID), p['ln_b'][i].reshape(1, _HID),
            whrT[j], whcT[j])
        for xp_part in xparts:
            xcur = xcur + xp_part

    chT16 = jnp.zeros((_HID, _XW), jnp.float32).at[:, :3].set(p['ch_w'].T)
    chb16 = jnp.zeros((1, _XW), jnp.float32).at[0, :3].set(p['ch_b'])
    hout, xd = _tc_epilogue(
        hcur,
        p['om_w1'].T, p['om_b1'].reshape(1, _HID),
        p['om_w2'].T, p['om_b2'].reshape(1, _HID),
        chT16, chb16)

    x_out = (xcur - x_init)[:_N, :3] + xd[:_N, :3]
    return (hout[:_N], x_out)
